# baseline (speedup 1.0000x reference)
import functools

import jax
import jax.numpy as jnp
from jax import lax
from jax.experimental import pallas as pl
from jax.experimental.pallas import tpu as pltpu
from jax.experimental.pallas import tpu_sc as plsc

N_SVC, N_POD, N_NODE = 10000, 50000, 5000
T, D, H = 8, 128, 128
E_SVC, E_IN, E_NP = 320000, 400000, 400000

P_SVC, P_POD, P_NODE = 10240, 50176, 5120
EP_SVC, EP_IN, EP_NP = 327680, 425984, 425984
NP_PASS = 32
CCOL = 32
NLANE = 16
NTILE = 16
NCORE = 2
SB = 128


def _sc_mesh():
    return plsc.VectorSubcoreMesh(core_axis_name="c", subcore_axis_name="s",
                                  num_cores=NCORE, num_subcores=NTILE)


def _degree_kernel(svc_s, svc_d, in_s, in_d, np_s, np_d):
    specs = [
        (svc_s, EP_SVC, P_SVC), (svc_d, EP_SVC, P_SVC),
        (in_s, EP_IN, P_POD), (in_d, EP_IN, P_NODE),
        (np_s, EP_NP, P_NODE), (np_d, EP_NP, P_POD),
    ]
    out_type = [jax.ShapeDtypeStruct((NCORE * NTILE, p), jnp.float32)
                for (_, _, p) in specs]
    scratch = [
        pltpu.VMEM((max(p for (_, _, p) in specs),), jnp.float32),
        pltpu.VMEM((max(e for (_, e, _) in specs) // (NCORE * NTILE) // 128, 128),
                   jnp.int32),
    ]

    def body(s1, s2, s3, s4, s5, s6, o1, o2, o3, o4, o5, o6, hist, idxc):
        wid = lax.axis_index("s") * NCORE + lax.axis_index("c")
        ones = jnp.ones((NLANE,), jnp.float32)
        for (arr, e_pad, nbins), out in zip(
                [(s1, EP_SVC, P_SVC), (s2, EP_SVC, P_SVC),
                 (s3, EP_IN, P_POD), (s4, EP_IN, P_NODE),
                 (s5, EP_NP, P_NODE), (s6, EP_NP, P_POD)],
                [o1, o2, o3, o4, o5, o6]):
            chunk = e_pad // (NCORE * NTILE)
            rows = chunk // 128

            def zero_b(i, _):
                hist[pl.ds(i * NLANE, NLANE)] = jnp.zeros((NLANE,), jnp.float32)
                return 0
            lax.fori_loop(0, nbins // NLANE, zero_b, 0)
            pltpu.sync_copy(arr.at[pl.ds(wid * rows, rows)], idxc.at[pl.ds(0, rows)])

            def scat(i, _):
                r = i // 8
                k = lax.rem(i, 8)
                idx = idxc[r, pl.ds(k * NLANE, NLANE)]
                plsc.addupdate_scatter(hist, [idx], ones)
                return 0
            lax.fori_loop(0, rows * 8, scat, 0)
            pltpu.sync_copy(hist.at[pl.ds(0, nbins)], out.at[wid])

    fn = pl.kernel(body, out_type=out_type, mesh=_sc_mesh(),
                   scratch_types=scratch,
                   compiler_params=pltpu.CompilerParams(
                       needs_layout_passes=False))
    return fn(svc_s.reshape(-1, 128), svc_d.reshape(-1, 128),
              in_s.reshape(-1, 128), in_d.reshape(-1, 128),
              np_s.reshape(-1, 128), np_d.reshape(-1, 128))


def _deg_finalize(hist_s, hist_d, pad_cnt):
    npd = hist_s.shape[1]
    bq = 512

    def body(hs_ref, hd_ref, rs_ref, rd_ref):
        i = pl.program_id(0)
        first = (lax.broadcasted_iota(jnp.int32, (bq,), 0) == 0) & (i == 0)
        ds = jnp.sum(hs_ref[...], axis=0)
        ds = jnp.maximum(ds - jnp.where(first, jnp.float32(pad_cnt), 0.0), 1.0)
        dd = jnp.maximum(jnp.sum(hd_ref[...], axis=0), 1.0)
        rs_ref[...] = jnp.broadcast_to(lax.rsqrt(ds)[:, None], (bq, 128))
        rd_ref[...] = jnp.broadcast_to(lax.rsqrt(dd)[:, None], (bq, 128))

    return pl.pallas_call(
        body,
        grid=(npd // bq,),
        in_specs=[pl.BlockSpec((NCORE * NTILE, bq), lambda i: (0, i))] * 2,
        out_specs=[pl.BlockSpec((bq, 128), lambda i: (i, 0))] * 2,
        out_shape=[jax.ShapeDtypeStruct((npd, 128), jnp.float32)] * 2,
    )(hist_s, hist_d)


def _prescale(feat, rs, npd, ccol):
    n = feat.shape[0]
    bn = 200
    nb = n // bn
    nch = D // ccol

    def body(f_ref, rs_ref, o_ref):
        xs = f_ref[...] * rs_ref[:, 0:1]
        for t in range(T):
            for c in range(nch):
                o_ref[t, c] = xs[:, t * D + c * ccol:t * D + (c + 1) * ccol]

    out = pl.pallas_call(
        body,
        grid=(nb,),
        in_specs=[
            pl.BlockSpec((bn, T * D), lambda i: (i, 0)),
            pl.BlockSpec((bn, 128), lambda i: (i, 0)),
        ],
        out_specs=pl.BlockSpec((T, nch, bn, ccol), lambda i: (0, 0, i, 0)),
        out_shape=jax.ShapeDtypeStruct((T, nch, npd, ccol), jnp.float32),
    )(feat.reshape(n, T * D), rs)
    return out.reshape(T * nch * npd, ccol)


def _aggregate(xs_flat, src2d, dst2d, n_src_pad, n_dst_pad, e_pad, ccol):
    chunk = e_pad // (NCORE * NTILE)
    b2 = chunk // SB
    rpt = n_dst_pad // NTILE
    nz = rpt // 64
    n_pass = (T * D) // ccol

    def body(xs_hbm, src_hbm, dst_hbm, out_hbm, srcc, dstc, rows, zbuf, acc, sem):
        cid = lax.axis_index("c")
        sid = lax.axis_index("s")
        row0 = (cid * NTILE + sid) * b2

        def zb(i, _):
            for q in range(ccol // NLANE):
                zbuf[i, pl.ds(q * NLANE, NLANE)] = jnp.zeros((NLANE,),
                                                             jnp.float32)
            return 0
        lax.fori_loop(0, 64, zb, 0)
        pltpu.sync_copy(src_hbm.at[pl.ds(row0, b2)], srcc)
        pltpu.sync_copy(dst_hbm.at[pl.ds(row0, b2)], dstc)

        def one_pass(p, _):
            @pl.when(p > 0)
            def _():
                def badd(i, _):
                    r = i // 8
                    k = lax.rem(i, 8)
                    srcc[r, pl.ds(k * NLANE, NLANE)] = (
                        srcc[r, pl.ds(k * NLANE, NLANE)] + n_src_pad)
                    return 0
                lax.fori_loop(0, b2 * 8, badd, 0)

            def zs(i, _):
                pltpu.sync_copy(zbuf, acc.at[pl.ds(sid * rpt + i * 64, 64)])
                return 0
            lax.fori_loop(0, nz, zs, 0)
            plsc.subcore_barrier()

            def burst(j, _):
                pltpu.async_copy(xs_hbm.at[srcc.at[j]], rows, sem).wait()
                pltpu.sync_copy(rows, acc.at[dstc.at[j]], add=True)
                return 0
            lax.fori_loop(0, b2, burst, 0)
            plsc.subcore_barrier()
            pltpu.sync_copy(acc.at[pl.ds(sid * rpt, rpt)],
                            out_hbm.at[cid, p, pl.ds(sid * rpt, rpt)])
            return 0
        lax.fori_loop(0, n_pass, one_pass, 0)

    fn = pl.kernel(
        body,
        out_type=jax.ShapeDtypeStruct((NCORE, n_pass, n_dst_pad, ccol),
                                      jnp.float32),
        mesh=_sc_mesh(),
        scratch_types=[
            pltpu.VMEM((b2, SB), jnp.int32),
            pltpu.VMEM((b2, SB), jnp.int32),
            pltpu.VMEM((SB, ccol), jnp.float32),
            pltpu.VMEM((64, ccol), jnp.float32),
            pltpu.VMEM_SHARED((n_dst_pad, ccol), jnp.float32),
            pltpu.SemaphoreType.DMA,
        ],
        compiler_params=pltpu.CompilerParams(needs_layout_passes=False,
                                             use_tc_tiling_on_sc=False),
    )
    return fn(xs_flat, src2d, dst2d)


def _reduce(a_part, w, b, rs_d, n_dst, n_dst_pad, ccol):
    bnd = 200
    nb = n_dst // bnd
    nch = D // ccol
    a5 = a_part.reshape(NCORE, T, nch, n_dst_pad, ccol)

    def body(a_ref, w_ref, b_ref, rs_ref, o_ref, acc, macc):
        t, i, c = pl.program_id(0), pl.program_id(1), pl.program_id(2)
        part = (a_ref[0, 0, 0] + a_ref[1, 0, 0]) @ w_ref[...]

        @pl.when(c == 0)
        def _():
            acc[...] = part

        @pl.when(c > 0)
        def _():
            acc[...] = acc[...] + part

        @pl.when(jnp.logical_and(i == 0, c == 0))
        def _():
            macc[...] = jnp.zeros_like(macc)

        @pl.when(c == nch - 1)
        def _():
            h = acc[...] * rs_ref[:, 0:1] + b_ref[...]
            lr = jnp.where(h > 0, h, 0.01 * h)
            macc[...] = macc[...] + jnp.sum(lr, axis=0, keepdims=True)

        @pl.when(jnp.logical_and(i == nb - 1, c == nch - 1))
        def _():
            o_ref[...] = jnp.broadcast_to(macc[...][:, None, :] * (1.0 / n_dst),
                                          (1, 8, H))

    return pl.pallas_call(
        body,
        grid=(T, nb, nch),
        in_specs=[
            pl.BlockSpec((NCORE, 1, 1, bnd, ccol), lambda t, i, c: (0, t, c, i, 0)),
            pl.BlockSpec((ccol, H), lambda t, i, c: (c, 0)),
            pl.BlockSpec((1, H), lambda t, i, c: (0, 0)),
            pl.BlockSpec((bnd, 128), lambda t, i, c: (i, 0)),
        ],
        out_specs=pl.BlockSpec((1, 8, H), lambda t, i, c: (t, 0, 0)),
        out_shape=jax.ShapeDtypeStruct((T, 8, H), jnp.float32),
        scratch_shapes=[pltpu.VMEM((bnd, H), jnp.float32),
                        pltpu.VMEM((1, H), jnp.float32)],
    )(a5, w, b.reshape(1, H), rs_d)[:, 0, :]


def _tail(m_svc, m_node, m_pod, lsw, lsb, lnw, lnb, lpw, lpb,
          wx0t, wh0t, b0, wx1t, wh1t, b1):
    def body(ms, mn, mp, lsw_r, lsb_r, lnw_r, lnb_r, lpw_r, lpb_r,
             wx0_r, wh0_r, b0_r, wx1_r, wh1_r, b1_r, o_ref, vbuf, seq0):
        vbuf[...] = (ms[...] @ lsw_r[...] + lsb_r[...]
                     + mn[...] @ lnw_r[...] + lnb_r[...]
                     + mp[...] @ lpw_r[...] + lpb_r[...]) * (1.0 / 3.0)

        def layer(src_ref, wx, wh, bb, dst_ref):
            def step(t, hc):
                h, c = hc
                z = src_ref[pl.ds(t, 1), :] @ wx + h @ wh + bb
                ii = jax.nn.sigmoid(z[:, 0:H])
                ff = jax.nn.sigmoid(z[:, H:2 * H])
                gg = jnp.tanh(z[:, 2 * H:3 * H])
                oo = jax.nn.sigmoid(z[:, 3 * H:4 * H])
                c = ff * c + ii * gg
                h = oo * jnp.tanh(c)
                dst_ref[pl.ds(t, 1), :] = h
                return (h, c)
            z0 = jnp.zeros((1, H), jnp.float32)
            lax.fori_loop(0, T, step, (z0, z0))

        layer(vbuf, wx0_r[...], wh0_r[...], b0_r[...], seq0)
        layer(seq0, wx1_r[...], wh1_r[...], b1_r[...], o_ref)

    return pl.pallas_call(
        body,
        out_shape=jax.ShapeDtypeStruct((T, H), jnp.float32),
        scratch_shapes=[pltpu.VMEM((T, H), jnp.float32),
                        pltpu.VMEM((T, H), jnp.float32)],
    )(m_svc, m_node, m_pod, lsw, lsb.reshape(1, H), lnw, lnb.reshape(1, H),
      lpw, lpb.reshape(1, H), wx0t, wh0t, b0.reshape(1, 4 * H),
      wx1t, wh1t, b1.reshape(1, 4 * H))


def kernel(svc_feat, pod_feat, node_feat, W_svc, b_svc, W_in, b_in, W_np, b_np,
           L_svc_w, L_svc_b, L_node_w, L_node_b, L_pod_w, L_pod_b,
           lstm_Wx0, lstm_Wh0, lstm_b0, lstm_Wx1, lstm_Wh1, lstm_b1,
           edge_svc_src, edge_svc_dst, edge_in_src, edge_in_dst,
           edge_np_src, edge_np_dst):
    def pad_e(src, dst, e_pad, n_dst):
        npad = e_pad - src.shape[0]
        return (jnp.concatenate([src, jnp.zeros((npad,), jnp.int32)]),
                jnp.concatenate([dst, jnp.full((npad,), n_dst, jnp.int32)]))

    svc_s, svc_d = pad_e(edge_svc_src, edge_svc_dst, EP_SVC, N_SVC)
    in_s, in_d = pad_e(edge_in_src, edge_in_dst, EP_IN, N_NODE)
    np_s, np_d = pad_e(edge_np_src, edge_np_dst, EP_NP, N_POD)

    h_svc_s, h_svc_d, h_in_s, h_in_d, h_np_s, h_np_d = _degree_kernel(
        svc_s, svc_d, in_s, in_d, np_s, np_d)

    rs_svc_s, rs_svc_d = _deg_finalize(h_svc_s, h_svc_d, EP_SVC - E_SVC)
    rs_pod_s, rs_pod_d = _deg_finalize(h_in_s, h_np_d, EP_IN - E_IN)
    rs_node_s, rs_node_d = _deg_finalize(h_np_s, h_in_d, EP_NP - E_NP)

    xs_svc = _prescale(svc_feat, rs_svc_s, P_SVC, 32)
    xs_pod = _prescale(pod_feat, rs_pod_s, P_POD, 32)
    xs_node = _prescale(node_feat, rs_node_s, P_NODE, 16)

    a_svc = _aggregate(xs_svc, svc_s.reshape(-1, SB), svc_d.reshape(-1, SB),
                       P_SVC, P_SVC, EP_SVC, 32)
    a_node = _aggregate(xs_pod, in_s.reshape(-1, SB), in_d.reshape(-1, SB),
                        P_POD, P_NODE, EP_IN, 32)
    a_pod = _aggregate(xs_node, np_s.reshape(-1, SB), np_d.reshape(-1, SB),
                       P_NODE, P_POD, EP_NP, 16)

    m_svc = _reduce(a_svc, W_svc, b_svc, rs_svc_d, N_SVC, P_SVC, 32)
    m_node = _reduce(a_node, W_in, b_in, rs_node_d, N_NODE, P_NODE, 32)
    m_pod = _reduce(a_pod, W_np, b_np, rs_pod_d, N_POD, P_POD, 16)

    return _tail(m_svc, m_node, m_pod, L_svc_w, L_svc_b, L_node_w, L_node_b,
                 L_pod_w, L_pod_b, lstm_Wx0.T, lstm_Wh0.T, lstm_b0,
                 lstm_Wx1.T, lstm_Wh1.T, lstm_b1)


# speedup vs baseline: 1.0448x; 1.0448x over previous
import functools

import jax
import jax.numpy as jnp
from jax import lax
from jax.experimental import pallas as pl
from jax.experimental.pallas import tpu as pltpu
from jax.experimental.pallas import tpu_sc as plsc

N_SVC, N_POD, N_NODE = 10000, 50000, 5000
T, D, H = 8, 128, 128
E_SVC, E_IN, E_NP = 320000, 400000, 400000

P_SVC, P_POD, P_NODE = 10240, 50176, 5120
EP_SVC, EP_IN, EP_NP = 327680, 425984, 425984
NP_PASS = 32
CCOL = 32
NLANE = 16
NTILE = 16
NCORE = 2
SB = 128


def _sc_mesh():
    return plsc.VectorSubcoreMesh(core_axis_name="c", subcore_axis_name="s",
                                  num_cores=NCORE, num_subcores=NTILE)


def _degree_kernel(svc_s, svc_d, in_s, in_d, np_s, np_d):
    specs = [
        (svc_s, EP_SVC, P_SVC), (svc_d, EP_SVC, P_SVC),
        (in_s, EP_IN, P_POD), (in_d, EP_IN, P_NODE),
        (np_s, EP_NP, P_NODE), (np_d, EP_NP, P_POD),
    ]
    out_type = [jax.ShapeDtypeStruct((NCORE * NTILE, p), jnp.float32)
                for (_, _, p) in specs]
    scratch = [
        pltpu.VMEM((max(p for (_, _, p) in specs),), jnp.float32),
        pltpu.VMEM((max(e for (_, e, _) in specs) // (NCORE * NTILE) // 128, 128),
                   jnp.int32),
    ]

    def body(s1, s2, s3, s4, s5, s6, o1, o2, o3, o4, o5, o6, hist, idxc):
        wid = lax.axis_index("s") * NCORE + lax.axis_index("c")
        ones = jnp.ones((NLANE,), jnp.float32)
        for (arr, e_pad, nbins), out in zip(
                [(s1, EP_SVC, P_SVC), (s2, EP_SVC, P_SVC),
                 (s3, EP_IN, P_POD), (s4, EP_IN, P_NODE),
                 (s5, EP_NP, P_NODE), (s6, EP_NP, P_POD)],
                [o1, o2, o3, o4, o5, o6]):
            chunk = e_pad // (NCORE * NTILE)
            rows = chunk // 128

            def zero_b(i, _):
                hist[pl.ds(i * NLANE, NLANE)] = jnp.zeros((NLANE,), jnp.float32)
                return 0
            lax.fori_loop(0, nbins // NLANE, zero_b, 0)
            pltpu.sync_copy(arr.at[pl.ds(wid * rows, rows)], idxc.at[pl.ds(0, rows)])

            def scat(i, _):
                r = i // 8
                k = lax.rem(i, 8)
                idx = idxc[r, pl.ds(k * NLANE, NLANE)]
                plsc.addupdate_scatter(hist, [idx], ones)
                return 0
            lax.fori_loop(0, rows * 8, scat, 0)
            pltpu.sync_copy(hist.at[pl.ds(0, nbins)], out.at[wid])

    fn = pl.kernel(body, out_type=out_type, mesh=_sc_mesh(),
                   scratch_types=scratch,
                   compiler_params=pltpu.CompilerParams(
                       needs_layout_passes=False))
    return fn(svc_s.reshape(-1, 128), svc_d.reshape(-1, 128),
              in_s.reshape(-1, 128), in_d.reshape(-1, 128),
              np_s.reshape(-1, 128), np_d.reshape(-1, 128))


def _deg_finalize(hist_s, hist_d, pad_cnt):
    npd = hist_s.shape[1]
    bq = 512

    def body(hs_ref, hd_ref, rs_ref, rd_ref):
        i = pl.program_id(0)
        first = (lax.broadcasted_iota(jnp.int32, (bq,), 0) == 0) & (i == 0)
        ds = jnp.sum(hs_ref[...], axis=0)
        ds = jnp.maximum(ds - jnp.where(first, jnp.float32(pad_cnt), 0.0), 1.0)
        dd = jnp.maximum(jnp.sum(hd_ref[...], axis=0), 1.0)
        rs_ref[...] = jnp.broadcast_to(lax.rsqrt(ds)[:, None], (bq, 128))
        rd_ref[...] = jnp.broadcast_to(lax.rsqrt(dd)[:, None], (bq, 128))

    return pl.pallas_call(
        body,
        grid=(npd // bq,),
        in_specs=[pl.BlockSpec((NCORE * NTILE, bq), lambda i: (0, i))] * 2,
        out_specs=[pl.BlockSpec((bq, 128), lambda i: (i, 0))] * 2,
        out_shape=[jax.ShapeDtypeStruct((npd, 128), jnp.float32)] * 2,
    )(hist_s, hist_d)


def _prescale(feat, rs, npd, ccol):
    n = feat.shape[0]
    bn = 200
    nb = n // bn
    nch = D // ccol

    def body(f_ref, rs_ref, o_ref):
        xs = f_ref[...] * rs_ref[:, 0:1]
        for t in range(T):
            for c in range(nch):
                o_ref[t, c] = xs[:, t * D + c * ccol:t * D + (c + 1) * ccol]

    out = pl.pallas_call(
        body,
        grid=(nb,),
        in_specs=[
            pl.BlockSpec((bn, T * D), lambda i: (i, 0)),
            pl.BlockSpec((bn, 128), lambda i: (i, 0)),
        ],
        out_specs=pl.BlockSpec((T, nch, bn, ccol), lambda i: (0, 0, i, 0)),
        out_shape=jax.ShapeDtypeStruct((T, nch, npd, ccol), jnp.float32),
    )(feat.reshape(n, T * D), rs)
    return out.reshape(T * nch * npd, ccol)


def _aggregate(xs_flat, src2d, dst2d, n_src_pad, n_dst_pad, e_pad, ccol):
    chunk = e_pad // (NCORE * NTILE)
    b2 = chunk // SB
    rpt = n_dst_pad // NTILE
    nz = rpt // 64
    n_pass = (T * D) // ccol

    def body(xs_hbm, src_hbm, dst_hbm, out_hbm, srcc, dstc, rows0, rows1,
             zbuf, acc, sem0, sem1):
        cid = lax.axis_index("c")
        sid = lax.axis_index("s")
        row0 = (cid * NTILE + sid) * b2

        def zb(i, _):
            for q in range(ccol // NLANE):
                zbuf[i, pl.ds(q * NLANE, NLANE)] = jnp.zeros((NLANE,),
                                                             jnp.float32)
            return 0
        lax.fori_loop(0, 64, zb, 0)
        pltpu.sync_copy(src_hbm.at[pl.ds(row0, b2)], srcc)
        pltpu.sync_copy(dst_hbm.at[pl.ds(row0, b2)], dstc)

        def one_pass(p, _):
            @pl.when(p > 0)
            def _():
                def badd(i, _):
                    r = i // 8
                    k = lax.rem(i, 8)
                    srcc[r, pl.ds(k * NLANE, NLANE)] = (
                        srcc[r, pl.ds(k * NLANE, NLANE)] + n_src_pad)
                    return 0
                lax.fori_loop(0, b2 * 8, badd, 0)

            def zs(i, _):
                pltpu.sync_copy(zbuf, acc.at[pl.ds(sid * rpt + i * 64, 64)])
                return 0
            lax.fori_loop(0, nz, zs, 0)
            plsc.subcore_barrier()

            pltpu.async_copy(xs_hbm.at[srcc.at[0]], rows0, sem0)

            def burst2(j2, _):
                j = 2 * j2
                pltpu.async_copy(xs_hbm.at[srcc.at[j + 1]], rows1, sem1)
                pltpu.make_async_copy(xs_hbm.at[srcc.at[j]], rows0, sem0).wait()
                pltpu.sync_copy(rows0, acc.at[dstc.at[j]], add=True)

                @pl.when(j2 < b2 // 2 - 1)
                def _():
                    pltpu.async_copy(xs_hbm.at[srcc.at[j + 2]], rows0, sem0)

                pltpu.make_async_copy(xs_hbm.at[srcc.at[j + 1]], rows1,
                                      sem1).wait()
                pltpu.sync_copy(rows1, acc.at[dstc.at[j + 1]], add=True)
                return 0
            lax.fori_loop(0, b2 // 2, burst2, 0)
            plsc.subcore_barrier()
            pltpu.sync_copy(acc.at[pl.ds(sid * rpt, rpt)],
                            out_hbm.at[cid, p, pl.ds(sid * rpt, rpt)])
            return 0
        lax.fori_loop(0, n_pass, one_pass, 0)

    fn = pl.kernel(
        body,
        out_type=jax.ShapeDtypeStruct((NCORE, n_pass, n_dst_pad, ccol),
                                      jnp.float32),
        mesh=_sc_mesh(),
        scratch_types=[
            pltpu.VMEM((b2, SB), jnp.int32),
            pltpu.VMEM((b2, SB), jnp.int32),
            pltpu.VMEM((SB, ccol), jnp.float32),
            pltpu.VMEM((SB, ccol), jnp.float32),
            pltpu.VMEM((64, ccol), jnp.float32),
            pltpu.VMEM_SHARED((n_dst_pad, ccol), jnp.float32),
            pltpu.SemaphoreType.DMA,
            pltpu.SemaphoreType.DMA,
        ],
        compiler_params=pltpu.CompilerParams(needs_layout_passes=False,
                                             use_tc_tiling_on_sc=False),
    )
    return fn(xs_flat, src2d, dst2d)


def _reduce(a_part, w, b, rs_d, n_dst, n_dst_pad, ccol):
    bnd = 200
    nb = n_dst // bnd
    nch = D // ccol
    a5 = a_part.reshape(NCORE, T, nch, n_dst_pad, ccol)

    def body(a_ref, w_ref, b_ref, rs_ref, o_ref, acc, macc):
        t, i, c = pl.program_id(0), pl.program_id(1), pl.program_id(2)
        part = (a_ref[0, 0, 0] + a_ref[1, 0, 0]) @ w_ref[...]

        @pl.when(c == 0)
        def _():
            acc[...] = part

        @pl.when(c > 0)
        def _():
            acc[...] = acc[...] + part

        @pl.when(jnp.logical_and(i == 0, c == 0))
        def _():
            macc[...] = jnp.zeros_like(macc)

        @pl.when(c == nch - 1)
        def _():
            h = acc[...] * rs_ref[:, 0:1] + b_ref[...]
            lr = jnp.where(h > 0, h, 0.01 * h)
            macc[...] = macc[...] + jnp.sum(lr, axis=0, keepdims=True)

        @pl.when(jnp.logical_and(i == nb - 1, c == nch - 1))
        def _():
            o_ref[...] = jnp.broadcast_to(macc[...][:, None, :] * (1.0 / n_dst),
                                          (1, 8, H))

    return pl.pallas_call(
        body,
        grid=(T, nb, nch),
        in_specs=[
            pl.BlockSpec((NCORE, 1, 1, bnd, ccol), lambda t, i, c: (0, t, c, i, 0)),
            pl.BlockSpec((ccol, H), lambda t, i, c: (c, 0)),
            pl.BlockSpec((1, H), lambda t, i, c: (0, 0)),
            pl.BlockSpec((bnd, 128), lambda t, i, c: (i, 0)),
        ],
        out_specs=pl.BlockSpec((1, 8, H), lambda t, i, c: (t, 0, 0)),
        out_shape=jax.ShapeDtypeStruct((T, 8, H), jnp.float32),
        scratch_shapes=[pltpu.VMEM((bnd, H), jnp.float32),
                        pltpu.VMEM((1, H), jnp.float32)],
    )(a5, w, b.reshape(1, H), rs_d)[:, 0, :]


def _tail(m_svc, m_node, m_pod, lsw, lsb, lnw, lnb, lpw, lpb,
          wx0t, wh0t, b0, wx1t, wh1t, b1):
    def body(ms, mn, mp, lsw_r, lsb_r, lnw_r, lnb_r, lpw_r, lpb_r,
             wx0_r, wh0_r, b0_r, wx1_r, wh1_r, b1_r, o_ref, vbuf, seq0):
        vbuf[...] = (ms[...] @ lsw_r[...] + lsb_r[...]
                     + mn[...] @ lnw_r[...] + lnb_r[...]
                     + mp[...] @ lpw_r[...] + lpb_r[...]) * (1.0 / 3.0)

        def layer(src_ref, wx, wh, bb, dst_ref):
            def step(t, hc):
                h, c = hc
                z = src_ref[pl.ds(t, 1), :] @ wx + h @ wh + bb
                ii = jax.nn.sigmoid(z[:, 0:H])
                ff = jax.nn.sigmoid(z[:, H:2 * H])
                gg = jnp.tanh(z[:, 2 * H:3 * H])
                oo = jax.nn.sigmoid(z[:, 3 * H:4 * H])
                c = ff * c + ii * gg
                h = oo * jnp.tanh(c)
                dst_ref[pl.ds(t, 1), :] = h
                return (h, c)
            z0 = jnp.zeros((1, H), jnp.float32)
            lax.fori_loop(0, T, step, (z0, z0))

        layer(vbuf, wx0_r[...], wh0_r[...], b0_r[...], seq0)
        layer(seq0, wx1_r[...], wh1_r[...], b1_r[...], o_ref)

    return pl.pallas_call(
        body,
        out_shape=jax.ShapeDtypeStruct((T, H), jnp.float32),
        scratch_shapes=[pltpu.VMEM((T, H), jnp.float32),
                        pltpu.VMEM((T, H), jnp.float32)],
    )(m_svc, m_node, m_pod, lsw, lsb.reshape(1, H), lnw, lnb.reshape(1, H),
      lpw, lpb.reshape(1, H), wx0t, wh0t, b0.reshape(1, 4 * H),
      wx1t, wh1t, b1.reshape(1, 4 * H))


def kernel(svc_feat, pod_feat, node_feat, W_svc, b_svc, W_in, b_in, W_np, b_np,
           L_svc_w, L_svc_b, L_node_w, L_node_b, L_pod_w, L_pod_b,
           lstm_Wx0, lstm_Wh0, lstm_b0, lstm_Wx1, lstm_Wh1, lstm_b1,
           edge_svc_src, edge_svc_dst, edge_in_src, edge_in_dst,
           edge_np_src, edge_np_dst):
    def pad_e(src, dst, e_pad, n_dst):
        npad = e_pad - src.shape[0]
        return (jnp.concatenate([src, jnp.zeros((npad,), jnp.int32)]),
                jnp.concatenate([dst, jnp.full((npad,), n_dst, jnp.int32)]))

    svc_s, svc_d = pad_e(edge_svc_src, edge_svc_dst, EP_SVC, N_SVC)
    in_s, in_d = pad_e(edge_in_src, edge_in_dst, EP_IN, N_NODE)
    np_s, np_d = pad_e(edge_np_src, edge_np_dst, EP_NP, N_POD)

    h_svc_s, h_svc_d, h_in_s, h_in_d, h_np_s, h_np_d = _degree_kernel(
        svc_s, svc_d, in_s, in_d, np_s, np_d)

    rs_svc_s, rs_svc_d = _deg_finalize(h_svc_s, h_svc_d, EP_SVC - E_SVC)
    rs_pod_s, rs_pod_d = _deg_finalize(h_in_s, h_np_d, EP_IN - E_IN)
    rs_node_s, rs_node_d = _deg_finalize(h_np_s, h_in_d, EP_NP - E_NP)

    xs_svc = _prescale(svc_feat, rs_svc_s, P_SVC, 32)
    xs_pod = _prescale(pod_feat, rs_pod_s, P_POD, 32)
    xs_node = _prescale(node_feat, rs_node_s, P_NODE, 16)

    a_svc = _aggregate(xs_svc, svc_s.reshape(-1, SB), svc_d.reshape(-1, SB),
                       P_SVC, P_SVC, EP_SVC, 32)
    a_node = _aggregate(xs_pod, in_s.reshape(-1, SB), in_d.reshape(-1, SB),
                        P_POD, P_NODE, EP_IN, 32)
    a_pod = _aggregate(xs_node, np_s.reshape(-1, SB), np_d.reshape(-1, SB),
                       P_NODE, P_POD, EP_NP, 16)

    m_svc = _reduce(a_svc, W_svc, b_svc, rs_svc_d, N_SVC, P_SVC, 32)
    m_node = _reduce(a_node, W_in, b_in, rs_node_d, N_NODE, P_NODE, 32)
    m_pod = _reduce(a_pod, W_np, b_np, rs_pod_d, N_POD, P_POD, 16)

    return _tail(m_svc, m_node, m_pod, L_svc_w, L_svc_b, L_node_w, L_node_b,
                 L_pod_w, L_pod_b, lstm_Wx0.T, lstm_Wh0.T, lstm_b0,
                 lstm_Wx1.T, lstm_Wh1.T, lstm_b1)


# speedup vs baseline: 1.9799x; 1.8951x over previous
import functools

import jax
import jax.numpy as jnp
from jax import lax
from jax.experimental import pallas as pl
from jax.experimental.pallas import tpu as pltpu
from jax.experimental.pallas import tpu_sc as plsc

N_SVC, N_POD, N_NODE = 10000, 50000, 5000
T, D, H = 8, 128, 128
E_SVC, E_IN, E_NP = 320000, 400000, 400000

P_SVC, P_POD, P_NODE = 10240, 50176, 5120
EP_SVC, EP_IN, EP_NP = 327680, 425984, 425984
NP_PASS = 32
CCOL = 32
NLANE = 16
NTILE = 16
NCORE = 2
SB = 128


def _sc_mesh():
    return plsc.VectorSubcoreMesh(core_axis_name="c", subcore_axis_name="s",
                                  num_cores=NCORE, num_subcores=NTILE)


def _degree_kernel(svc_s, svc_d, in_s, in_d, np_s, np_d):
    specs = [
        (svc_s, EP_SVC, P_SVC), (svc_d, EP_SVC, P_SVC),
        (in_s, EP_IN, P_POD), (in_d, EP_IN, P_NODE),
        (np_s, EP_NP, P_NODE), (np_d, EP_NP, P_POD),
    ]
    out_type = [jax.ShapeDtypeStruct((NCORE * NTILE, p), jnp.float32)
                for (_, _, p) in specs]
    scratch = [
        pltpu.VMEM((max(p for (_, _, p) in specs),), jnp.float32),
        pltpu.VMEM((max(e for (_, e, _) in specs) // (NCORE * NTILE) // 128, 128),
                   jnp.int32),
    ]

    def body(s1, s2, s3, s4, s5, s6, o1, o2, o3, o4, o5, o6, hist, idxc):
        wid = lax.axis_index("s") * NCORE + lax.axis_index("c")
        ones = jnp.ones((NLANE,), jnp.float32)
        for (arr, e_pad, nbins), out in zip(
                [(s1, EP_SVC, P_SVC), (s2, EP_SVC, P_SVC),
                 (s3, EP_IN, P_POD), (s4, EP_IN, P_NODE),
                 (s5, EP_NP, P_NODE), (s6, EP_NP, P_POD)],
                [o1, o2, o3, o4, o5, o6]):
            chunk = e_pad // (NCORE * NTILE)
            rows = chunk // 128

            def zero_b(i, _):
                hist[pl.ds(i * NLANE, NLANE)] = jnp.zeros((NLANE,), jnp.float32)
                return 0
            lax.fori_loop(0, nbins // NLANE, zero_b, 0)
            pltpu.sync_copy(arr.at[pl.ds(wid * rows, rows)], idxc.at[pl.ds(0, rows)])

            def scat(i, _):
                r = i // 8
                k = lax.rem(i, 8)
                idx = idxc[r, pl.ds(k * NLANE, NLANE)]
                plsc.addupdate_scatter(hist, [idx], ones)
                return 0
            lax.fori_loop(0, rows * 8, scat, 0)
            pltpu.sync_copy(hist.at[pl.ds(0, nbins)], out.at[wid])

    fn = pl.kernel(body, out_type=out_type, mesh=_sc_mesh(),
                   scratch_types=scratch,
                   compiler_params=pltpu.CompilerParams(
                       needs_layout_passes=False))
    return fn(svc_s.reshape(-1, 128), svc_d.reshape(-1, 128),
              in_s.reshape(-1, 128), in_d.reshape(-1, 128),
              np_s.reshape(-1, 128), np_d.reshape(-1, 128))


def _deg_finalize(hist_s, hist_d, pad_cnt):
    npd = hist_s.shape[1]
    bq = 512

    def body(hs_ref, hd_ref, rs_ref, rd_ref):
        i = pl.program_id(0)
        first = (lax.broadcasted_iota(jnp.int32, (bq,), 0) == 0) & (i == 0)
        ds = jnp.sum(hs_ref[...], axis=0)
        ds = jnp.maximum(ds - jnp.where(first, jnp.float32(pad_cnt), 0.0), 1.0)
        dd = jnp.maximum(jnp.sum(hd_ref[...], axis=0), 1.0)
        rs_ref[...] = jnp.broadcast_to(lax.rsqrt(ds)[:, None], (bq, 128))
        rd_ref[...] = jnp.broadcast_to(lax.rsqrt(dd)[:, None], (bq, 128))

    return pl.pallas_call(
        body,
        grid=(npd // bq,),
        in_specs=[pl.BlockSpec((NCORE * NTILE, bq), lambda i: (0, i))] * 2,
        out_specs=[pl.BlockSpec((bq, 128), lambda i: (i, 0))] * 2,
        out_shape=[jax.ShapeDtypeStruct((npd, 128), jnp.float32)] * 2,
    )(hist_s, hist_d)


def _prescale(feat, rs, npd, ccol):
    n = feat.shape[0]
    bn = 512
    nb = npd // bn
    nch = D // ccol

    def body(f_ref, rs_ref, o_ref):
        xs = (f_ref[...] * rs_ref[:, 0:1]).astype(jnp.bfloat16)
        for t in range(T):
            for c in range(nch):
                o_ref[t, c] = xs[:, t * D + c * ccol:t * D + (c + 1) * ccol]

    out = pl.pallas_call(
        body,
        grid=(nb,),
        in_specs=[
            pl.BlockSpec((bn, T * D), lambda i: (i, 0)),
            pl.BlockSpec((bn, 128), lambda i: (i, 0)),
        ],
        out_specs=pl.BlockSpec((T, nch, bn, ccol), lambda i: (0, 0, i, 0)),
        out_shape=jax.ShapeDtypeStruct((T, nch, npd, ccol), jnp.bfloat16),
    )(feat.reshape(n, T * D), rs)
    return out.reshape(T * nch * npd, ccol)


def _aggregate(xs_flat, src2d, dst2d, n_src_pad, n_dst_pad, e_pad, ccol):
    chunk = e_pad // (NCORE * NTILE)
    b2 = chunk // SB
    rpt = n_dst_pad // NTILE
    nz = rpt // 64
    n_pass = (T * D) // ccol

    def body(xs_hbm, src_hbm, dst_hbm, out_hbm, srcc, dstc, rows0, rows1,
             zbuf, acc, sem0, sem1):
        cid = lax.axis_index("c")
        sid = lax.axis_index("s")
        row0 = (cid * NTILE + sid) * b2

        def zb(i, _):
            zbuf[i, pl.ds(0, ccol)] = jnp.zeros((ccol,), jnp.bfloat16)
            return 0
        lax.fori_loop(0, 64, zb, 0)
        pltpu.sync_copy(src_hbm.at[pl.ds(row0, b2)], srcc)
        pltpu.sync_copy(dst_hbm.at[pl.ds(row0, b2)], dstc)

        def one_pass(p, _):
            @pl.when(p > 0)
            def _():
                def badd(i, _):
                    r = i // 8
                    k = lax.rem(i, 8)
                    srcc[r, pl.ds(k * NLANE, NLANE)] = (
                        srcc[r, pl.ds(k * NLANE, NLANE)] + n_src_pad)
                    return 0
                lax.fori_loop(0, b2 * 8, badd, 0)

            def zs(i, _):
                pltpu.sync_copy(zbuf, acc.at[pl.ds(sid * rpt + i * 64, 64)])
                return 0
            lax.fori_loop(0, nz, zs, 0)
            plsc.subcore_barrier()

            pltpu.async_copy(xs_hbm.at[srcc.at[0]], rows0, sem0)

            def burst2(j2, _):
                j = 2 * j2
                pltpu.async_copy(xs_hbm.at[srcc.at[j + 1]], rows1, sem1)
                pltpu.make_async_copy(xs_hbm.at[srcc.at[j]], rows0, sem0).wait()
                pltpu.sync_copy(rows0, acc.at[dstc.at[j]], add=True)

                @pl.when(j2 < b2 // 2 - 1)
                def _():
                    pltpu.async_copy(xs_hbm.at[srcc.at[j + 2]], rows0, sem0)

                pltpu.make_async_copy(xs_hbm.at[srcc.at[j + 1]], rows1,
                                      sem1).wait()
                pltpu.sync_copy(rows1, acc.at[dstc.at[j + 1]], add=True)
                return 0
            lax.fori_loop(0, b2 // 2, burst2, 0)
            plsc.subcore_barrier()
            pltpu.sync_copy(acc.at[pl.ds(sid * rpt, rpt)],
                            out_hbm.at[cid, p, pl.ds(sid * rpt, rpt)])
            return 0
        lax.fori_loop(0, n_pass, one_pass, 0)

    fn = pl.kernel(
        body,
        out_type=jax.ShapeDtypeStruct((NCORE, n_pass, n_dst_pad, ccol),
                                      jnp.bfloat16),
        mesh=_sc_mesh(),
        scratch_types=[
            pltpu.VMEM((b2, SB), jnp.int32),
            pltpu.VMEM((b2, SB), jnp.int32),
            pltpu.VMEM((SB, ccol), jnp.bfloat16),
            pltpu.VMEM((SB, ccol), jnp.bfloat16),
            pltpu.VMEM((64, ccol), jnp.bfloat16),
            pltpu.VMEM_SHARED((n_dst_pad, ccol), jnp.bfloat16),
            pltpu.SemaphoreType.DMA,
            pltpu.SemaphoreType.DMA,
        ],
        compiler_params=pltpu.CompilerParams(needs_layout_passes=False,
                                             use_tc_tiling_on_sc=False),
    )
    return fn(xs_flat, src2d, dst2d)


def _reduce(a_part, w, b, rs_d, n_dst, n_dst_pad, ccol):
    bnd = 512
    nb = n_dst_pad // bnd
    nch = D // ccol
    a5 = a_part.reshape(NCORE, T, nch, n_dst_pad, ccol)

    def body(a_ref, w_ref, b_ref, rs_ref, o_ref, acc, macc):
        t, i, c = pl.program_id(0), pl.program_id(1), pl.program_id(2)
        part = (a_ref[0, 0, 0].astype(jnp.float32)
                + a_ref[1, 0, 0].astype(jnp.float32)) @ w_ref[...]

        @pl.when(c == 0)
        def _():
            acc[...] = part

        @pl.when(c > 0)
        def _():
            acc[...] = acc[...] + part

        @pl.when(jnp.logical_and(i == 0, c == 0))
        def _():
            macc[...] = jnp.zeros_like(macc)

        @pl.when(c == nch - 1)
        def _():
            h = acc[...] * rs_ref[:, 0:1] + b_ref[...]
            lr = jnp.where(h > 0, h, 0.01 * h)
            row = (lax.broadcasted_iota(jnp.int32, (bnd, 1), 0)
                   + i * bnd)
            lr = jnp.where(row < n_dst, lr, 0.0)
            macc[...] = macc[...] + jnp.sum(lr, axis=0, keepdims=True)

        @pl.when(jnp.logical_and(i == nb - 1, c == nch - 1))
        def _():
            o_ref[...] = jnp.broadcast_to(macc[...][:, None, :] * (1.0 / n_dst),
                                          (1, 8, H))

    return pl.pallas_call(
        body,
        grid=(T, nb, nch),
        in_specs=[
            pl.BlockSpec((NCORE, 1, 1, bnd, ccol), lambda t, i, c: (0, t, c, i, 0)),
            pl.BlockSpec((ccol, H), lambda t, i, c: (c, 0)),
            pl.BlockSpec((1, H), lambda t, i, c: (0, 0)),
            pl.BlockSpec((bnd, 128), lambda t, i, c: (i, 0)),
        ],
        out_specs=pl.BlockSpec((1, 8, H), lambda t, i, c: (t, 0, 0)),
        out_shape=jax.ShapeDtypeStruct((T, 8, H), jnp.float32),
        scratch_shapes=[pltpu.VMEM((bnd, H), jnp.float32),
                        pltpu.VMEM((1, H), jnp.float32)],
    )(a5, w, b.reshape(1, H), rs_d)[:, 0, :]


def _tail(m_svc, m_node, m_pod, lsw, lsb, lnw, lnb, lpw, lpb,
          wx0t, wh0t, b0, wx1t, wh1t, b1):
    def body(ms, mn, mp, lsw_r, lsb_r, lnw_r, lnb_r, lpw_r, lpb_r,
             wx0_r, wh0_r, b0_r, wx1_r, wh1_r, b1_r, o_ref, vbuf, seq0):
        vbuf[...] = (ms[...] @ lsw_r[...] + lsb_r[...]
                     + mn[...] @ lnw_r[...] + lnb_r[...]
                     + mp[...] @ lpw_r[...] + lpb_r[...]) * (1.0 / 3.0)

        def layer(src_ref, wx, wh, bb, dst_ref):
            def step(t, hc):
                h, c = hc
                z = src_ref[pl.ds(t, 1), :] @ wx + h @ wh + bb
                ii = jax.nn.sigmoid(z[:, 0:H])
                ff = jax.nn.sigmoid(z[:, H:2 * H])
                gg = jnp.tanh(z[:, 2 * H:3 * H])
                oo = jax.nn.sigmoid(z[:, 3 * H:4 * H])
                c = ff * c + ii * gg
                h = oo * jnp.tanh(c)
                dst_ref[pl.ds(t, 1), :] = h
                return (h, c)
            z0 = jnp.zeros((1, H), jnp.float32)
            lax.fori_loop(0, T, step, (z0, z0))

        layer(vbuf, wx0_r[...], wh0_r[...], b0_r[...], seq0)
        layer(seq0, wx1_r[...], wh1_r[...], b1_r[...], o_ref)

    return pl.pallas_call(
        body,
        out_shape=jax.ShapeDtypeStruct((T, H), jnp.float32),
        scratch_shapes=[pltpu.VMEM((T, H), jnp.float32),
                        pltpu.VMEM((T, H), jnp.float32)],
    )(m_svc, m_node, m_pod, lsw, lsb.reshape(1, H), lnw, lnb.reshape(1, H),
      lpw, lpb.reshape(1, H), wx0t, wh0t, b0.reshape(1, 4 * H),
      wx1t, wh1t, b1.reshape(1, 4 * H))


def kernel(svc_feat, pod_feat, node_feat, W_svc, b_svc, W_in, b_in, W_np, b_np,
           L_svc_w, L_svc_b, L_node_w, L_node_b, L_pod_w, L_pod_b,
           lstm_Wx0, lstm_Wh0, lstm_b0, lstm_Wx1, lstm_Wh1, lstm_b1,
           edge_svc_src, edge_svc_dst, edge_in_src, edge_in_dst,
           edge_np_src, edge_np_dst):
    def pad_e(src, dst, e_pad, n_dst):
        npad = e_pad - src.shape[0]
        return (jnp.concatenate([src, jnp.zeros((npad,), jnp.int32)]),
                jnp.concatenate([dst, jnp.full((npad,), n_dst, jnp.int32)]))

    svc_s, svc_d = pad_e(edge_svc_src, edge_svc_dst, EP_SVC, N_SVC)
    in_s, in_d = pad_e(edge_in_src, edge_in_dst, EP_IN, N_NODE)
    np_s, np_d = pad_e(edge_np_src, edge_np_dst, EP_NP, N_POD)

    h_svc_s, h_svc_d, h_in_s, h_in_d, h_np_s, h_np_d = _degree_kernel(
        svc_s, svc_d, in_s, in_d, np_s, np_d)

    rs_svc_s, rs_svc_d = _deg_finalize(h_svc_s, h_svc_d, EP_SVC - E_SVC)
    rs_pod_s, rs_pod_d = _deg_finalize(h_in_s, h_np_d, EP_IN - E_IN)
    rs_node_s, rs_node_d = _deg_finalize(h_np_s, h_in_d, EP_NP - E_NP)

    xs_svc = _prescale(svc_feat, rs_svc_s, P_SVC, 32)
    xs_pod = _prescale(pod_feat, rs_pod_s, P_POD, 32)
    xs_node = _prescale(node_feat, rs_node_s, P_NODE, 32)

    a_svc = _aggregate(xs_svc, svc_s.reshape(-1, SB), svc_d.reshape(-1, SB),
                       P_SVC, P_SVC, EP_SVC, 32)
    a_node = _aggregate(xs_pod, in_s.reshape(-1, SB), in_d.reshape(-1, SB),
                        P_POD, P_NODE, EP_IN, 32)
    a_pod = _aggregate(xs_node, np_s.reshape(-1, SB), np_d.reshape(-1, SB),
                       P_NODE, P_POD, EP_NP, 32)

    m_svc = _reduce(a_svc, W_svc, b_svc, rs_svc_d, N_SVC, P_SVC, 32)
    m_node = _reduce(a_node, W_in, b_in, rs_node_d, N_NODE, P_NODE, 32)
    m_pod = _reduce(a_pod, W_np, b_np, rs_pod_d, N_POD, P_POD, 32)

    return _tail(m_svc, m_node, m_pod, L_svc_w, L_svc_b, L_node_w, L_node_b,
                 L_pod_w, L_pod_b, lstm_Wx0.T, lstm_Wh0.T, lstm_b0,
                 lstm_Wx1.T, lstm_Wh1.T, lstm_b1)


# speedup vs baseline: 3.8858x; 1.9626x over previous
import functools

import jax
import jax.numpy as jnp
from jax import lax
from jax.experimental import pallas as pl
from jax.experimental.pallas import tpu as pltpu
from jax.experimental.pallas import tpu_sc as plsc

N_SVC, N_POD, N_NODE = 10000, 50000, 5000
T, D, H = 8, 128, 128
E_SVC, E_IN, E_NP = 320000, 400000, 400000

P_SVC, P_POD, P_NODE = 10240, 50176, 5120
EP_SVC, EP_IN, EP_NP = 327680, 425984, 425984
NP_PASS = 32
CCOL = 32
NLANE = 16
NTILE = 16
NCORE = 2
SB = 128


def _sc_mesh():
    return plsc.VectorSubcoreMesh(core_axis_name="c", subcore_axis_name="s",
                                  num_cores=NCORE, num_subcores=NTILE)


def _degree_kernel(svc_s, svc_d, in_s, in_d, np_s, np_d):
    specs = [
        (svc_s, EP_SVC, P_SVC), (svc_d, EP_SVC, P_SVC),
        (in_s, EP_IN, P_POD), (in_d, EP_IN, P_NODE),
        (np_s, EP_NP, P_NODE), (np_d, EP_NP, P_POD),
    ]
    out_type = [jax.ShapeDtypeStruct((NCORE * NTILE, p), jnp.float32)
                for (_, _, p) in specs]
    scratch = [
        pltpu.VMEM((max(p for (_, _, p) in specs),), jnp.float32),
        pltpu.VMEM((max(e for (_, e, _) in specs) // (NCORE * NTILE) // 128, 128),
                   jnp.int32),
    ]

    def body(s1, s2, s3, s4, s5, s6, o1, o2, o3, o4, o5, o6, hist, idxc):
        wid = lax.axis_index("s") * NCORE + lax.axis_index("c")
        ones = jnp.ones((NLANE,), jnp.float32)
        for (arr, e_pad, nbins), out in zip(
                [(s1, EP_SVC, P_SVC), (s2, EP_SVC, P_SVC),
                 (s3, EP_IN, P_POD), (s4, EP_IN, P_NODE),
                 (s5, EP_NP, P_NODE), (s6, EP_NP, P_POD)],
                [o1, o2, o3, o4, o5, o6]):
            chunk = e_pad // (NCORE * NTILE)
            rows = chunk // 128

            def zero_b(i, _):
                hist[pl.ds(i * NLANE, NLANE)] = jnp.zeros((NLANE,), jnp.float32)
                return 0
            lax.fori_loop(0, nbins // NLANE, zero_b, 0)
            pltpu.sync_copy(arr.at[pl.ds(wid * rows, rows)], idxc.at[pl.ds(0, rows)])

            def scat(i, _):
                r = i // 8
                k = lax.rem(i, 8)
                idx = idxc[r, pl.ds(k * NLANE, NLANE)]
                plsc.addupdate_scatter(hist, [idx], ones)
                return 0
            lax.fori_loop(0, rows * 8, scat, 0)
            pltpu.sync_copy(hist.at[pl.ds(0, nbins)], out.at[wid])

    fn = pl.kernel(body, out_type=out_type, mesh=_sc_mesh(),
                   scratch_types=scratch,
                   compiler_params=pltpu.CompilerParams(
                       needs_layout_passes=False))
    return fn(svc_s.reshape(-1, 128), svc_d.reshape(-1, 128),
              in_s.reshape(-1, 128), in_d.reshape(-1, 128),
              np_s.reshape(-1, 128), np_d.reshape(-1, 128))


def _deg_finalize(hist_s, hist_d, pad_cnt):
    npd = hist_s.shape[1]
    bq = 512

    def body(hs_ref, hd_ref, rs_ref, rd_ref):
        i = pl.program_id(0)
        first = (lax.broadcasted_iota(jnp.int32, (bq,), 0) < 128) & (i == 0)
        ds = jnp.sum(hs_ref[...], axis=0)
        ds = jnp.maximum(ds - jnp.where(first, jnp.float32(pad_cnt // 128),
                                        0.0), 1.0)
        dd = jnp.maximum(jnp.sum(hd_ref[...], axis=0), 1.0)
        rs_ref[...] = jnp.broadcast_to(lax.rsqrt(ds)[:, None], (bq, 128))
        rd_ref[...] = jnp.broadcast_to(lax.rsqrt(dd)[:, None], (bq, 128))

    return pl.pallas_call(
        body,
        grid=(npd // bq,),
        in_specs=[pl.BlockSpec((NCORE * NTILE, bq), lambda i: (0, i))] * 2,
        out_specs=[pl.BlockSpec((bq, 128), lambda i: (i, 0))] * 2,
        out_shape=[jax.ShapeDtypeStruct((npd, 128), jnp.float32)] * 2,
    )(hist_s, hist_d)


def _prescale(feat, rs, npd, ccol):
    n = feat.shape[0]
    bn = 512
    nb = npd // bn
    nch = D // ccol

    def body(f_ref, rs_ref, o_ref):
        xs = (f_ref[...] * rs_ref[:, 0:1]).astype(jnp.bfloat16)
        for t in range(T):
            for c in range(nch):
                o_ref[t, c] = xs[:, t * D + c * ccol:t * D + (c + 1) * ccol]

    out = pl.pallas_call(
        body,
        grid=(nb,),
        in_specs=[
            pl.BlockSpec((bn, T * D), lambda i: (i, 0)),
            pl.BlockSpec((bn, 128), lambda i: (i, 0)),
        ],
        out_specs=pl.BlockSpec((T, nch, bn, ccol), lambda i: (0, 0, i, 0)),
        out_shape=jax.ShapeDtypeStruct((T, nch, npd, ccol), jnp.bfloat16),
    )(feat.reshape(n, T * D), rs)
    return out.reshape(T * nch * npd, ccol)


def _aggregate(xs_flat, src2d, dst2d, n_src_pad, n_dst_pad, e_pad, ccol):
    chunk = e_pad // (NCORE * NTILE)
    b2 = chunk // SB
    rpt = n_dst_pad // NTILE
    nz = rpt // 64
    n_pass = (T * D) // ccol

    def body(xs_hbm, src_hbm, dst_hbm, out_hbm, srcc, dstc, rows0, rows1,
             zbuf, acc, sem0, sem1):
        cid = lax.axis_index("c")
        sid = lax.axis_index("s")
        row0 = (cid * NTILE + sid) * b2

        def zb(i, _):
            zbuf[i, pl.ds(0, ccol)] = jnp.zeros((ccol,), jnp.bfloat16)
            return 0
        lax.fori_loop(0, 64, zb, 0)
        pltpu.sync_copy(src_hbm.at[pl.ds(row0, b2)], srcc)
        pltpu.sync_copy(dst_hbm.at[pl.ds(row0, b2)], dstc)

        def one_pass(p, _):
            @pl.when(p > 0)
            def _():
                def badd(i, _):
                    r = i // 8
                    k = lax.rem(i, 8)
                    srcc[r, pl.ds(k * NLANE, NLANE)] = (
                        srcc[r, pl.ds(k * NLANE, NLANE)] + n_src_pad)
                    return 0
                lax.fori_loop(0, b2 * 8, badd, 0)

            def zs(i, _):
                pltpu.sync_copy(zbuf, acc.at[pl.ds(sid * rpt + i * 64, 64)])
                return 0
            lax.fori_loop(0, nz, zs, 0)
            plsc.subcore_barrier()

            pltpu.async_copy(xs_hbm.at[srcc.at[0]], rows0, sem0)

            def burst2(j2, _):
                j = 2 * j2
                pltpu.async_copy(xs_hbm.at[srcc.at[j + 1]], rows1, sem1)
                pltpu.make_async_copy(xs_hbm.at[srcc.at[j]], rows0, sem0).wait()
                pltpu.sync_copy(rows0, acc.at[dstc.at[j]], add=True)

                @pl.when(j2 < b2 // 2 - 1)
                def _():
                    pltpu.async_copy(xs_hbm.at[srcc.at[j + 2]], rows0, sem0)

                pltpu.make_async_copy(xs_hbm.at[srcc.at[j + 1]], rows1,
                                      sem1).wait()
                pltpu.sync_copy(rows1, acc.at[dstc.at[j + 1]], add=True)
                return 0
            lax.fori_loop(0, b2 // 2, burst2, 0)
            plsc.subcore_barrier()
            pltpu.sync_copy(acc.at[pl.ds(sid * rpt, rpt)],
                            out_hbm.at[cid, p, pl.ds(sid * rpt, rpt)])
            return 0
        lax.fori_loop(0, n_pass, one_pass, 0)

    fn = pl.kernel(
        body,
        out_type=jax.ShapeDtypeStruct((NCORE, n_pass, n_dst_pad, ccol),
                                      jnp.bfloat16),
        mesh=_sc_mesh(),
        scratch_types=[
            pltpu.VMEM((b2, SB), jnp.int32),
            pltpu.VMEM((b2, SB), jnp.int32),
            pltpu.VMEM((SB, ccol), jnp.bfloat16),
            pltpu.VMEM((SB, ccol), jnp.bfloat16),
            pltpu.VMEM((64, ccol), jnp.bfloat16),
            pltpu.VMEM_SHARED((n_dst_pad, ccol), jnp.bfloat16),
            pltpu.SemaphoreType.DMA,
            pltpu.SemaphoreType.DMA,
        ],
        compiler_params=pltpu.CompilerParams(needs_layout_passes=False,
                                             use_tc_tiling_on_sc=False),
    )
    return fn(xs_flat, src2d, dst2d)


def _reduce(a_part, w, b, rs_d, n_dst, n_dst_pad, ccol):
    bnd = 512
    nb = n_dst_pad // bnd
    nch = D // ccol
    a5 = a_part.reshape(NCORE, T, nch, n_dst_pad, ccol)

    def body(a_ref, w_ref, b_ref, rs_ref, o_ref, acc, macc):
        t, i, c = pl.program_id(0), pl.program_id(1), pl.program_id(2)
        part = (a_ref[0, 0, 0].astype(jnp.float32)
                + a_ref[1, 0, 0].astype(jnp.float32)) @ w_ref[...]

        @pl.when(c == 0)
        def _():
            acc[...] = part

        @pl.when(c > 0)
        def _():
            acc[...] = acc[...] + part

        @pl.when(jnp.logical_and(i == 0, c == 0))
        def _():
            macc[...] = jnp.zeros_like(macc)

        @pl.when(c == nch - 1)
        def _():
            h = acc[...] * rs_ref[:, 0:1] + b_ref[...]
            lr = jnp.where(h > 0, h, 0.01 * h)
            row = (lax.broadcasted_iota(jnp.int32, (bnd, 1), 0)
                   + i * bnd)
            lr = jnp.where(row < n_dst, lr, 0.0)
            macc[...] = macc[...] + jnp.sum(lr, axis=0, keepdims=True)

        @pl.when(jnp.logical_and(i == nb - 1, c == nch - 1))
        def _():
            o_ref[...] = jnp.broadcast_to(macc[...][:, None, :] * (1.0 / n_dst),
                                          (1, 8, H))

    return pl.pallas_call(
        body,
        grid=(T, nb, nch),
        in_specs=[
            pl.BlockSpec((NCORE, 1, 1, bnd, ccol), lambda t, i, c: (0, t, c, i, 0)),
            pl.BlockSpec((ccol, H), lambda t, i, c: (c, 0)),
            pl.BlockSpec((1, H), lambda t, i, c: (0, 0)),
            pl.BlockSpec((bnd, 128), lambda t, i, c: (i, 0)),
        ],
        out_specs=pl.BlockSpec((1, 8, H), lambda t, i, c: (t, 0, 0)),
        out_shape=jax.ShapeDtypeStruct((T, 8, H), jnp.float32),
        scratch_shapes=[pltpu.VMEM((bnd, H), jnp.float32),
                        pltpu.VMEM((1, H), jnp.float32)],
    )(a5, w, b.reshape(1, H), rs_d)[:, 0, :]


def _tail(m_svc, m_node, m_pod, lsw, lsb, lnw, lnb, lpw, lpb,
          wx0t, wh0t, b0, wx1t, wh1t, b1):
    def body(ms, mn, mp, lsw_r, lsb_r, lnw_r, lnb_r, lpw_r, lpb_r,
             wx0_r, wh0_r, b0_r, wx1_r, wh1_r, b1_r, o_ref, vbuf, seq0):
        vbuf[...] = (ms[...] @ lsw_r[...] + lsb_r[...]
                     + mn[...] @ lnw_r[...] + lnb_r[...]
                     + mp[...] @ lpw_r[...] + lpb_r[...]) * (1.0 / 3.0)

        def layer(src_ref, wx, wh, bb, dst_ref):
            def step(t, hc):
                h, c = hc
                z = src_ref[pl.ds(t, 1), :] @ wx + h @ wh + bb
                ii = jax.nn.sigmoid(z[:, 0:H])
                ff = jax.nn.sigmoid(z[:, H:2 * H])
                gg = jnp.tanh(z[:, 2 * H:3 * H])
                oo = jax.nn.sigmoid(z[:, 3 * H:4 * H])
                c = ff * c + ii * gg
                h = oo * jnp.tanh(c)
                dst_ref[pl.ds(t, 1), :] = h
                return (h, c)
            z0 = jnp.zeros((1, H), jnp.float32)
            lax.fori_loop(0, T, step, (z0, z0))

        layer(vbuf, wx0_r[...], wh0_r[...], b0_r[...], seq0)
        layer(seq0, wx1_r[...], wh1_r[...], b1_r[...], o_ref)

    return pl.pallas_call(
        body,
        out_shape=jax.ShapeDtypeStruct((T, H), jnp.float32),
        scratch_shapes=[pltpu.VMEM((T, H), jnp.float32),
                        pltpu.VMEM((T, H), jnp.float32)],
    )(m_svc, m_node, m_pod, lsw, lsb.reshape(1, H), lnw, lnb.reshape(1, H),
      lpw, lpb.reshape(1, H), wx0t, wh0t, b0.reshape(1, 4 * H),
      wx1t, wh1t, b1.reshape(1, 4 * H))


def kernel(svc_feat, pod_feat, node_feat, W_svc, b_svc, W_in, b_in, W_np, b_np,
           L_svc_w, L_svc_b, L_node_w, L_node_b, L_pod_w, L_pod_b,
           lstm_Wx0, lstm_Wh0, lstm_b0, lstm_Wx1, lstm_Wh1, lstm_b1,
           edge_svc_src, edge_svc_dst, edge_in_src, edge_in_dst,
           edge_np_src, edge_np_dst):
    def pad_e(src, dst, e_pad, n_dst, n_dst_pad):
        npad = e_pad - src.shape[0]
        ar = jnp.arange(npad, dtype=jnp.int32)
        return (jnp.concatenate([src, ar % 128]),
                jnp.concatenate([dst, n_dst + ar % (n_dst_pad - n_dst)]))

    svc_s, svc_d = pad_e(edge_svc_src, edge_svc_dst, EP_SVC, N_SVC, P_SVC)
    in_s, in_d = pad_e(edge_in_src, edge_in_dst, EP_IN, N_NODE, P_NODE)
    np_s, np_d = pad_e(edge_np_src, edge_np_dst, EP_NP, N_POD, P_POD)

    h_svc_s, h_svc_d, h_in_s, h_in_d, h_np_s, h_np_d = _degree_kernel(
        svc_s, svc_d, in_s, in_d, np_s, np_d)

    rs_svc_s, rs_svc_d = _deg_finalize(h_svc_s, h_svc_d, EP_SVC - E_SVC)
    rs_pod_s, rs_pod_d = _deg_finalize(h_in_s, h_np_d, EP_IN - E_IN)
    rs_node_s, rs_node_d = _deg_finalize(h_np_s, h_in_d, EP_NP - E_NP)

    xs_svc = _prescale(svc_feat, rs_svc_s, P_SVC, 32)
    xs_pod = _prescale(pod_feat, rs_pod_s, P_POD, 32)
    xs_node = _prescale(node_feat, rs_node_s, P_NODE, 32)

    a_svc = _aggregate(xs_svc, svc_s.reshape(-1, SB), svc_d.reshape(-1, SB),
                       P_SVC, P_SVC, EP_SVC, 32)
    a_node = _aggregate(xs_pod, in_s.reshape(-1, SB), in_d.reshape(-1, SB),
                        P_POD, P_NODE, EP_IN, 32)
    a_pod = _aggregate(xs_node, np_s.reshape(-1, SB), np_d.reshape(-1, SB),
                       P_NODE, P_POD, EP_NP, 32)

    m_svc = _reduce(a_svc, W_svc, b_svc, rs_svc_d, N_SVC, P_SVC, 32)
    m_node = _reduce(a_node, W_in, b_in, rs_node_d, N_NODE, P_NODE, 32)
    m_pod = _reduce(a_pod, W_np, b_np, rs_pod_d, N_POD, P_POD, 32)

    return _tail(m_svc, m_node, m_pod, L_svc_w, L_svc_b, L_node_w, L_node_b,
                 L_pod_w, L_pod_b, lstm_Wx0.T, lstm_Wh0.T, lstm_b0,
                 lstm_Wx1.T, lstm_Wh1.T, lstm_b1)


# speedup vs baseline: 4.2841x; 1.1025x over previous
import functools

import jax
import jax.numpy as jnp
from jax import lax
from jax.experimental import pallas as pl
from jax.experimental.pallas import tpu as pltpu
from jax.experimental.pallas import tpu_sc as plsc

N_SVC, N_POD, N_NODE = 10000, 50000, 5000
T, D, H = 8, 128, 128
E_SVC, E_IN, E_NP = 320000, 400000, 400000

P_SVC, P_POD, P_NODE = 10240, 50176, 5120
EP_SVC, EP_IN, EP_NP = 327680, 425984, 425984
NP_PASS = 32
CCOL = 32
NLANE = 16
NTILE = 16
NCORE = 2
SB = 128


def _sc_mesh():
    return plsc.VectorSubcoreMesh(core_axis_name="c", subcore_axis_name="s",
                                  num_cores=NCORE, num_subcores=NTILE)


def _degree_kernel(svc_s, svc_d, in_s, in_d, np_s, np_d):
    specs = [
        (svc_s, EP_SVC, P_SVC), (svc_d, EP_SVC, P_SVC),
        (in_s, EP_IN, P_POD), (in_d, EP_IN, P_NODE),
        (np_s, EP_NP, P_NODE), (np_d, EP_NP, P_POD),
    ]
    out_type = [jax.ShapeDtypeStruct((NCORE * NTILE, p), jnp.float32)
                for (_, _, p) in specs]
    scratch = [
        pltpu.VMEM((max(p for (_, _, p) in specs),), jnp.float32),
        pltpu.VMEM((max(e for (_, e, _) in specs) // (NCORE * NTILE) // 128, 128),
                   jnp.int32),
    ]

    def body(s1, s2, s3, s4, s5, s6, o1, o2, o3, o4, o5, o6, hist, idxc):
        wid = lax.axis_index("s") * NCORE + lax.axis_index("c")
        ones = jnp.ones((NLANE,), jnp.float32)
        for (arr, e_pad, nbins), out in zip(
                [(s1, EP_SVC, P_SVC), (s2, EP_SVC, P_SVC),
                 (s3, EP_IN, P_POD), (s4, EP_IN, P_NODE),
                 (s5, EP_NP, P_NODE), (s6, EP_NP, P_POD)],
                [o1, o2, o3, o4, o5, o6]):
            chunk = e_pad // (NCORE * NTILE)
            rows = chunk // 128

            def zero_b(i, _):
                hist[pl.ds(i * NLANE, NLANE)] = jnp.zeros((NLANE,), jnp.float32)
                return 0
            lax.fori_loop(0, nbins // NLANE, zero_b, 0)
            pltpu.sync_copy(arr.at[pl.ds(wid * rows, rows)], idxc.at[pl.ds(0, rows)])

            def scat(i, _):
                r = i // 8
                k = lax.rem(i, 8)
                idx = idxc[r, pl.ds(k * NLANE, NLANE)]
                plsc.addupdate_scatter(hist, [idx], ones)
                return 0
            lax.fori_loop(0, rows * 8, scat, 0)
            pltpu.sync_copy(hist.at[pl.ds(0, nbins)], out.at[wid])

    fn = pl.kernel(body, out_type=out_type, mesh=_sc_mesh(),
                   scratch_types=scratch,
                   compiler_params=pltpu.CompilerParams(
                       needs_layout_passes=False))
    return fn(svc_s.reshape(-1, 128), svc_d.reshape(-1, 128),
              in_s.reshape(-1, 128), in_d.reshape(-1, 128),
              np_s.reshape(-1, 128), np_d.reshape(-1, 128))


def _deg_finalize(hist_s, hist_d, pad_cnt):
    npd = hist_s.shape[1]
    bq = 512

    def body(hs_ref, hd_ref, rs_ref, rd_ref):
        i = pl.program_id(0)
        first = (lax.broadcasted_iota(jnp.int32, (bq,), 0) < 128) & (i == 0)
        ds = jnp.sum(hs_ref[...], axis=0)
        ds = jnp.maximum(ds - jnp.where(first, jnp.float32(pad_cnt // 128),
                                        0.0), 1.0)
        dd = jnp.maximum(jnp.sum(hd_ref[...], axis=0), 1.0)
        rs_ref[...] = jnp.broadcast_to(lax.rsqrt(ds)[:, None], (bq, 8))
        rd_ref[...] = jnp.broadcast_to(lax.rsqrt(dd)[:, None], (bq, 8))

    return pl.pallas_call(
        body,
        grid=(npd // bq,),
        in_specs=[pl.BlockSpec((NCORE * NTILE, bq), lambda i: (0, i))] * 2,
        out_specs=[pl.BlockSpec((bq, 8), lambda i: (i, 0))] * 2,
        out_shape=[jax.ShapeDtypeStruct((npd, 8), jnp.float32)] * 2,
    )(hist_s, hist_d)


def _prescale(feat, rs, npd, ccol):
    n = feat.shape[0]
    bn = 512
    nb = npd // bn
    nch = D // ccol

    def body(f_ref, rs_ref, o_ref):
        xs = (f_ref[...] * rs_ref[:, 0:1]).astype(jnp.bfloat16)
        for t in range(T):
            for c in range(nch):
                o_ref[t, c] = xs[:, t * D + c * ccol:t * D + (c + 1) * ccol]

    out = pl.pallas_call(
        body,
        grid=(nb,),
        in_specs=[
            pl.BlockSpec((bn, T * D), lambda i: (i, 0)),
            pl.BlockSpec((bn, 8), lambda i: (i, 0)),
        ],
        out_specs=pl.BlockSpec((T, nch, bn, ccol), lambda i: (0, 0, i, 0)),
        out_shape=jax.ShapeDtypeStruct((T, nch, npd, ccol), jnp.bfloat16),
    )(feat.reshape(n, T * D), rs)
    return out.reshape(T * nch * npd, ccol)


def _aggregate(xs_flat, src2d, dst2d, n_src_pad, n_dst_pad, e_pad, ccol):
    chunk = e_pad // (NCORE * NTILE)
    b2 = chunk // SB
    rpt = n_dst_pad // NTILE
    nz = rpt // 64
    n_pass = (T * D) // ccol

    def body(xs_hbm, src_hbm, dst_hbm, out_hbm, srcc, dstc, rows0, rows1,
             zbuf, acc, sem0, sem1):
        cid = lax.axis_index("c")
        sid = lax.axis_index("s")
        row0 = (cid * NTILE + sid) * b2

        def zb(i, _):
            for q in range(ccol // 32):
                zbuf[i, pl.ds(q * 32, 32)] = jnp.zeros((32,), jnp.bfloat16)
            return 0
        lax.fori_loop(0, 64, zb, 0)
        pltpu.sync_copy(src_hbm.at[pl.ds(row0, b2)], srcc)
        pltpu.sync_copy(dst_hbm.at[pl.ds(row0, b2)], dstc)

        def one_pass(p, _):
            @pl.when(p > 0)
            def _():
                def badd(i, _):
                    r = i // 8
                    k = lax.rem(i, 8)
                    srcc[r, pl.ds(k * NLANE, NLANE)] = (
                        srcc[r, pl.ds(k * NLANE, NLANE)] + n_src_pad)
                    return 0
                lax.fori_loop(0, b2 * 8, badd, 0)

            def zs(i, _):
                pltpu.sync_copy(zbuf, acc.at[pl.ds(sid * rpt + i * 64, 64)])
                return 0
            lax.fori_loop(0, nz, zs, 0)
            plsc.subcore_barrier()

            pltpu.async_copy(xs_hbm.at[srcc.at[0]], rows0, sem0)

            def burst2(j2, _):
                j = 2 * j2
                pltpu.async_copy(xs_hbm.at[srcc.at[j + 1]], rows1, sem1)
                pltpu.make_async_copy(xs_hbm.at[srcc.at[j]], rows0, sem0).wait()
                pltpu.sync_copy(rows0, acc.at[dstc.at[j]], add=True)

                @pl.when(j2 < b2 // 2 - 1)
                def _():
                    pltpu.async_copy(xs_hbm.at[srcc.at[j + 2]], rows0, sem0)

                pltpu.make_async_copy(xs_hbm.at[srcc.at[j + 1]], rows1,
                                      sem1).wait()
                pltpu.sync_copy(rows1, acc.at[dstc.at[j + 1]], add=True)
                return 0
            lax.fori_loop(0, b2 // 2, burst2, 0)
            plsc.subcore_barrier()
            pltpu.sync_copy(acc.at[pl.ds(sid * rpt, rpt)],
                            out_hbm.at[cid, p, pl.ds(sid * rpt, rpt)])
            return 0
        lax.fori_loop(0, n_pass, one_pass, 0)

    fn = pl.kernel(
        body,
        out_type=jax.ShapeDtypeStruct((NCORE, n_pass, n_dst_pad, ccol),
                                      jnp.bfloat16),
        mesh=_sc_mesh(),
        scratch_types=[
            pltpu.VMEM((b2, SB), jnp.int32),
            pltpu.VMEM((b2, SB), jnp.int32),
            pltpu.VMEM((SB, ccol), jnp.bfloat16),
            pltpu.VMEM((SB, ccol), jnp.bfloat16),
            pltpu.VMEM((64, ccol), jnp.bfloat16),
            pltpu.VMEM_SHARED((n_dst_pad, ccol), jnp.bfloat16),
            pltpu.SemaphoreType.DMA,
            pltpu.SemaphoreType.DMA,
        ],
        compiler_params=pltpu.CompilerParams(needs_layout_passes=False,
                                             use_tc_tiling_on_sc=False),
    )
    return fn(xs_flat, src2d, dst2d)


def _reduce(a_part, w, b, rs_d, n_dst, n_dst_pad, ccol):
    bnd = 512
    nb = n_dst_pad // bnd
    nch = D // ccol
    a5 = a_part.reshape(NCORE, T, nch, n_dst_pad, ccol)

    def body(a_ref, w_ref, b_ref, rs_ref, o_ref, acc, macc):
        t, i, c = pl.program_id(0), pl.program_id(1), pl.program_id(2)
        part = (a_ref[0, 0, 0].astype(jnp.float32)
                + a_ref[1, 0, 0].astype(jnp.float32)) @ w_ref[...]

        @pl.when(c == 0)
        def _():
            acc[...] = part

        @pl.when(c > 0)
        def _():
            acc[...] = acc[...] + part

        @pl.when(jnp.logical_and(i == 0, c == 0))
        def _():
            macc[...] = jnp.zeros_like(macc)

        @pl.when(c == nch - 1)
        def _():
            h = acc[...] * rs_ref[:, 0:1] + b_ref[...]
            lr = jnp.where(h > 0, h, 0.01 * h)
            row = (lax.broadcasted_iota(jnp.int32, (bnd, 1), 0)
                   + i * bnd)
            lr = jnp.where(row < n_dst, lr, 0.0)
            macc[...] = macc[...] + jnp.sum(lr, axis=0, keepdims=True)

        @pl.when(jnp.logical_and(i == nb - 1, c == nch - 1))
        def _():
            o_ref[...] = jnp.broadcast_to(macc[...][:, None, :] * (1.0 / n_dst),
                                          (1, 8, H))

    return pl.pallas_call(
        body,
        grid=(T, nb, nch),
        in_specs=[
            pl.BlockSpec((NCORE, 1, 1, bnd, ccol), lambda t, i, c: (0, t, c, i, 0)),
            pl.BlockSpec((ccol, H), lambda t, i, c: (c, 0)),
            pl.BlockSpec((1, H), lambda t, i, c: (0, 0)),
            pl.BlockSpec((bnd, 8), lambda t, i, c: (i, 0)),
        ],
        out_specs=pl.BlockSpec((1, 8, H), lambda t, i, c: (t, 0, 0)),
        out_shape=jax.ShapeDtypeStruct((T, 8, H), jnp.float32),
        scratch_shapes=[pltpu.VMEM((bnd, H), jnp.float32),
                        pltpu.VMEM((1, H), jnp.float32)],
    )(a5, w, b.reshape(1, H), rs_d)[:, 0, :]


def _tail(m_svc, m_node, m_pod, lsw, lsb, lnw, lnb, lpw, lpb,
          wx0t, wh0t, b0, wx1t, wh1t, b1):
    def body(ms, mn, mp, lsw_r, lsb_r, lnw_r, lnb_r, lpw_r, lpb_r,
             wx0_r, wh0_r, b0_r, wx1_r, wh1_r, b1_r, o_ref, vbuf, seq0):
        vbuf[...] = (ms[...] @ lsw_r[...] + lsb_r[...]
                     + mn[...] @ lnw_r[...] + lnb_r[...]
                     + mp[...] @ lpw_r[...] + lpb_r[...]) * (1.0 / 3.0)

        def layer(src_ref, wx, wh, bb, dst_ref):
            def step(t, hc):
                h, c = hc
                z = src_ref[pl.ds(t, 1), :] @ wx + h @ wh + bb
                ii = jax.nn.sigmoid(z[:, 0:H])
                ff = jax.nn.sigmoid(z[:, H:2 * H])
                gg = jnp.tanh(z[:, 2 * H:3 * H])
                oo = jax.nn.sigmoid(z[:, 3 * H:4 * H])
                c = ff * c + ii * gg
                h = oo * jnp.tanh(c)
                dst_ref[pl.ds(t, 1), :] = h
                return (h, c)
            z0 = jnp.zeros((1, H), jnp.float32)
            lax.fori_loop(0, T, step, (z0, z0))

        layer(vbuf, wx0_r[...], wh0_r[...], b0_r[...], seq0)
        layer(seq0, wx1_r[...], wh1_r[...], b1_r[...], o_ref)

    return pl.pallas_call(
        body,
        out_shape=jax.ShapeDtypeStruct((T, H), jnp.float32),
        scratch_shapes=[pltpu.VMEM((T, H), jnp.float32),
                        pltpu.VMEM((T, H), jnp.float32)],
    )(m_svc, m_node, m_pod, lsw, lsb.reshape(1, H), lnw, lnb.reshape(1, H),
      lpw, lpb.reshape(1, H), wx0t, wh0t, b0.reshape(1, 4 * H),
      wx1t, wh1t, b1.reshape(1, 4 * H))


def kernel(svc_feat, pod_feat, node_feat, W_svc, b_svc, W_in, b_in, W_np, b_np,
           L_svc_w, L_svc_b, L_node_w, L_node_b, L_pod_w, L_pod_b,
           lstm_Wx0, lstm_Wh0, lstm_b0, lstm_Wx1, lstm_Wh1, lstm_b1,
           edge_svc_src, edge_svc_dst, edge_in_src, edge_in_dst,
           edge_np_src, edge_np_dst):
    def pad_e(src, dst, e_pad, n_dst, n_dst_pad):
        npad = e_pad - src.shape[0]
        ar = jnp.arange(npad, dtype=jnp.int32)
        return (jnp.concatenate([src, ar % 128]),
                jnp.concatenate([dst, n_dst + ar % (n_dst_pad - n_dst)]))

    svc_s, svc_d = pad_e(edge_svc_src, edge_svc_dst, EP_SVC, N_SVC, P_SVC)
    in_s, in_d = pad_e(edge_in_src, edge_in_dst, EP_IN, N_NODE, P_NODE)
    np_s, np_d = pad_e(edge_np_src, edge_np_dst, EP_NP, N_POD, P_POD)

    h_svc_s, h_svc_d, h_in_s, h_in_d, h_np_s, h_np_d = _degree_kernel(
        svc_s, svc_d, in_s, in_d, np_s, np_d)

    rs_svc_s, rs_svc_d = _deg_finalize(h_svc_s, h_svc_d, EP_SVC - E_SVC)
    rs_pod_s, rs_pod_d = _deg_finalize(h_in_s, h_np_d, EP_IN - E_IN)
    rs_node_s, rs_node_d = _deg_finalize(h_np_s, h_in_d, EP_NP - E_NP)

    xs_svc = _prescale(svc_feat, rs_svc_s, P_SVC, 64)
    xs_pod = _prescale(pod_feat, rs_pod_s, P_POD, 64)
    xs_node = _prescale(node_feat, rs_node_s, P_NODE, 32)

    a_svc = _aggregate(xs_svc, svc_s.reshape(-1, SB), svc_d.reshape(-1, SB),
                       P_SVC, P_SVC, EP_SVC, 64)
    a_node = _aggregate(xs_pod, in_s.reshape(-1, SB), in_d.reshape(-1, SB),
                        P_POD, P_NODE, EP_IN, 64)
    a_pod = _aggregate(xs_node, np_s.reshape(-1, SB), np_d.reshape(-1, SB),
                       P_NODE, P_POD, EP_NP, 32)

    m_svc = _reduce(a_svc, W_svc, b_svc, rs_svc_d, N_SVC, P_SVC, 64)
    m_node = _reduce(a_node, W_in, b_in, rs_node_d, N_NODE, P_NODE, 64)
    m_pod = _reduce(a_pod, W_np, b_np, rs_pod_d, N_POD, P_POD, 32)

    return _tail(m_svc, m_node, m_pod, L_svc_w, L_svc_b, L_node_w, L_node_b,
                 L_pod_w, L_pod_b, lstm_Wx0.T, lstm_Wh0.T, lstm_b0,
                 lstm_Wx1.T, lstm_Wh1.T, lstm_b1)


# speedup vs baseline: 4.5388x; 1.0594x over previous
import functools

import jax
import jax.numpy as jnp
from jax import lax
from jax.experimental import pallas as pl
from jax.experimental.pallas import tpu as pltpu
from jax.experimental.pallas import tpu_sc as plsc

N_SVC, N_POD, N_NODE = 10000, 50000, 5000
T, D, H = 8, 128, 128
E_SVC, E_IN, E_NP = 320000, 400000, 400000

P_SVC, P_POD, P_NODE = 10240, 50176, 5120
EP_SVC, EP_IN, EP_NP = 327680, 425984, 425984
NP_PASS = 32
CCOL = 32
NLANE = 16
NTILE = 16
NCORE = 2
SB = 128


def _sc_mesh():
    return plsc.VectorSubcoreMesh(core_axis_name="c", subcore_axis_name="s",
                                  num_cores=NCORE, num_subcores=NTILE)


def _degree_kernel(svc_s, svc_d, in_s, in_d, np_s, np_d):
    specs = [
        (svc_s, EP_SVC, P_SVC), (svc_d, EP_SVC, P_SVC),
        (in_s, EP_IN, P_POD), (in_d, EP_IN, P_NODE),
        (np_s, EP_NP, P_NODE), (np_d, EP_NP, P_POD),
    ]
    out_type = [jax.ShapeDtypeStruct((NCORE * NTILE, p), jnp.float32)
                for (_, _, p) in specs]
    scratch = [
        pltpu.VMEM((max(p for (_, _, p) in specs),), jnp.float32),
        pltpu.VMEM((max(e for (_, e, _) in specs) // (NCORE * NTILE) // 128, 128),
                   jnp.int32),
    ]

    def body(s1, s2, s3, s4, s5, s6, o1, o2, o3, o4, o5, o6, hist, idxc):
        wid = lax.axis_index("s") * NCORE + lax.axis_index("c")
        ones = jnp.ones((NLANE,), jnp.float32)
        for (arr, e_pad, nbins), out in zip(
                [(s1, EP_SVC, P_SVC), (s2, EP_SVC, P_SVC),
                 (s3, EP_IN, P_POD), (s4, EP_IN, P_NODE),
                 (s5, EP_NP, P_NODE), (s6, EP_NP, P_POD)],
                [o1, o2, o3, o4, o5, o6]):
            chunk = e_pad // (NCORE * NTILE)
            rows = chunk // 128

            def zero_b(i, _):
                hist[pl.ds(i * NLANE, NLANE)] = jnp.zeros((NLANE,), jnp.float32)
                return 0
            lax.fori_loop(0, nbins // NLANE, zero_b, 0)
            pltpu.sync_copy(arr.at[pl.ds(wid * rows, rows)], idxc.at[pl.ds(0, rows)])

            def scat(i, _):
                r = i // 8
                k = lax.rem(i, 8)
                idx = idxc[r, pl.ds(k * NLANE, NLANE)]
                plsc.addupdate_scatter(hist, [idx], ones)
                return 0
            lax.fori_loop(0, rows * 8, scat, 0)
            pltpu.sync_copy(hist.at[pl.ds(0, nbins)], out.at[wid])

    fn = pl.kernel(body, out_type=out_type, mesh=_sc_mesh(),
                   scratch_types=scratch,
                   compiler_params=pltpu.CompilerParams(
                       needs_layout_passes=False))
    return fn(svc_s.reshape(-1, 128), svc_d.reshape(-1, 128),
              in_s.reshape(-1, 128), in_d.reshape(-1, 128),
              np_s.reshape(-1, 128), np_d.reshape(-1, 128))


def _deg_finalize(hist_s, hist_d, pad_cnt):
    npd = hist_s.shape[1]
    bq = 512

    def body(hs_ref, hd_ref, rs_ref, rd_ref):
        i = pl.program_id(0)
        first = (lax.broadcasted_iota(jnp.int32, (bq,), 0) < 128) & (i == 0)
        ds = jnp.sum(hs_ref[...], axis=0)
        ds = jnp.maximum(ds - jnp.where(first, jnp.float32(pad_cnt // 128),
                                        0.0), 1.0)
        dd = jnp.maximum(jnp.sum(hd_ref[...], axis=0), 1.0)
        rs_ref[...] = jnp.broadcast_to(lax.rsqrt(ds)[:, None], (bq, 8))
        rd_ref[...] = jnp.broadcast_to(lax.rsqrt(dd)[:, None], (bq, 8))

    return pl.pallas_call(
        body,
        grid=(npd // bq,),
        in_specs=[pl.BlockSpec((NCORE * NTILE, bq), lambda i: (0, i))] * 2,
        out_specs=[pl.BlockSpec((bq, 8), lambda i: (i, 0))] * 2,
        out_shape=[jax.ShapeDtypeStruct((npd, 8), jnp.float32)] * 2,
    )(hist_s, hist_d)


def _prescale(feat, rs, npd, ccol):
    n = feat.shape[0]
    bn = 512
    nb = npd // bn
    nch = D // ccol

    def body(f_ref, rs_ref, o_ref):
        xs = (f_ref[...] * rs_ref[:, 0:1]).astype(jnp.bfloat16)
        for t in range(T):
            for c in range(nch):
                o_ref[t, c] = xs[:, t * D + c * ccol:t * D + (c + 1) * ccol]

    out = pl.pallas_call(
        body,
        grid=(nb,),
        in_specs=[
            pl.BlockSpec((bn, T * D), lambda i: (i, 0)),
            pl.BlockSpec((bn, 8), lambda i: (i, 0)),
        ],
        out_specs=pl.BlockSpec((T, nch, bn, ccol), lambda i: (0, 0, i, 0)),
        out_shape=jax.ShapeDtypeStruct((T, nch, npd, ccol), jnp.bfloat16),
    )(feat.reshape(n, T * D), rs)
    return out.reshape(T * nch * npd, ccol)


def _aggregate(xs_flat, src2d, dst2d, n_src_pad, n_dst_pad, e_pad, ccol):
    chunk = e_pad // (NCORE * NTILE)
    b2 = chunk // SB
    rpt = n_dst_pad // NTILE
    nz = rpt // 64
    n_pass = (T * D) // ccol

    def body(xs_hbm, src_hbm, dst_hbm, out_hbm, srcc, dstc, rows0, rows1,
             zbuf, acc, sem0, sem1):
        cid = lax.axis_index("c")
        sid = lax.axis_index("s")
        row0 = (cid * NTILE + sid) * b2

        def zb(i, _):
            for q in range(ccol // 32):
                zbuf[i, pl.ds(q * 32, 32)] = jnp.zeros((32,), jnp.bfloat16)
            return 0
        lax.fori_loop(0, 64, zb, 0)
        pltpu.sync_copy(src_hbm.at[pl.ds(row0, b2)], srcc)
        pltpu.sync_copy(dst_hbm.at[pl.ds(row0, b2)], dstc)

        def one_pass(p, _):
            @pl.when(p > 0)
            def _():
                def badd(i, _):
                    r = i // 8
                    k = lax.rem(i, 8)
                    srcc[r, pl.ds(k * NLANE, NLANE)] = (
                        srcc[r, pl.ds(k * NLANE, NLANE)] + n_src_pad)
                    return 0
                lax.fori_loop(0, b2 * 8, badd, 0)

            def zs(i, _):
                pltpu.sync_copy(zbuf, acc.at[pl.ds(sid * rpt + i * 64, 64)])
                return 0
            lax.fori_loop(0, nz, zs, 0)
            plsc.subcore_barrier()

            pltpu.async_copy(xs_hbm.at[srcc.at[0]], rows0, sem0)

            def burst2(j2, _):
                j = 2 * j2
                pltpu.async_copy(xs_hbm.at[srcc.at[j + 1]], rows1, sem1)
                pltpu.make_async_copy(xs_hbm.at[srcc.at[j]], rows0, sem0).wait()
                pltpu.sync_copy(rows0, acc.at[dstc.at[j]], add=True)

                @pl.when(j2 < b2 // 2 - 1)
                def _():
                    pltpu.async_copy(xs_hbm.at[srcc.at[j + 2]], rows0, sem0)

                pltpu.make_async_copy(xs_hbm.at[srcc.at[j + 1]], rows1,
                                      sem1).wait()
                pltpu.sync_copy(rows1, acc.at[dstc.at[j + 1]], add=True)
                return 0
            lax.fori_loop(0, b2 // 2, burst2, 0)
            plsc.subcore_barrier()
            pltpu.sync_copy(acc.at[pl.ds(sid * rpt, rpt)],
                            out_hbm.at[cid, p, pl.ds(sid * rpt, rpt)])
            return 0
        lax.fori_loop(0, n_pass, one_pass, 0)

    fn = pl.kernel(
        body,
        out_type=jax.ShapeDtypeStruct((NCORE, n_pass, n_dst_pad, ccol),
                                      jnp.bfloat16),
        mesh=_sc_mesh(),
        scratch_types=[
            pltpu.VMEM((b2, SB), jnp.int32),
            pltpu.VMEM((b2, SB), jnp.int32),
            pltpu.VMEM((SB, ccol), jnp.bfloat16),
            pltpu.VMEM((SB, ccol), jnp.bfloat16),
            pltpu.VMEM((64, ccol), jnp.bfloat16),
            pltpu.VMEM_SHARED((n_dst_pad, ccol), jnp.bfloat16),
            pltpu.SemaphoreType.DMA,
            pltpu.SemaphoreType.DMA,
        ],
        compiler_params=pltpu.CompilerParams(needs_layout_passes=False,
                                             use_tc_tiling_on_sc=False),
    )
    return fn(xs_flat, src2d, dst2d)


def _reduce(a_part, w, b, rs_d, n_dst, n_dst_pad, ccol):
    bnd = 512
    nb = n_dst_pad // bnd
    nch = D // ccol
    a5 = a_part.reshape(NCORE, T, nch, n_dst_pad, ccol)

    def body(a_ref, w_ref, b_ref, rs_ref, o_ref, acc, macc):
        t, i, c = pl.program_id(0), pl.program_id(1), pl.program_id(2)
        part = (a_ref[0, 0, 0].astype(jnp.float32)
                + a_ref[1, 0, 0].astype(jnp.float32)) @ w_ref[...]

        @pl.when(c == 0)
        def _():
            acc[...] = part

        @pl.when(c > 0)
        def _():
            acc[...] = acc[...] + part

        @pl.when(jnp.logical_and(i == 0, c == 0))
        def _():
            macc[...] = jnp.zeros_like(macc)

        @pl.when(c == nch - 1)
        def _():
            h = acc[...] * rs_ref[:, 0:1] + b_ref[...]
            lr = jnp.where(h > 0, h, 0.01 * h)
            row = (lax.broadcasted_iota(jnp.int32, (bnd, 1), 0)
                   + i * bnd)
            lr = jnp.where(row < n_dst, lr, 0.0)
            macc[...] = macc[...] + jnp.sum(lr, axis=0, keepdims=True)

        @pl.when(jnp.logical_and(i == nb - 1, c == nch - 1))
        def _():
            o_ref[...] = jnp.broadcast_to(macc[...][:, None, :] * (1.0 / n_dst),
                                          (1, 8, H))

    return pl.pallas_call(
        body,
        grid=(T, nb, nch),
        in_specs=[
            pl.BlockSpec((NCORE, 1, 1, bnd, ccol), lambda t, i, c: (0, t, c, i, 0)),
            pl.BlockSpec((ccol, H), lambda t, i, c: (c, 0)),
            pl.BlockSpec((1, H), lambda t, i, c: (0, 0)),
            pl.BlockSpec((bnd, 8), lambda t, i, c: (i, 0)),
        ],
        out_specs=pl.BlockSpec((1, 8, H), lambda t, i, c: (t, 0, 0)),
        out_shape=jax.ShapeDtypeStruct((T, 8, H), jnp.float32),
        scratch_shapes=[pltpu.VMEM((bnd, H), jnp.float32),
                        pltpu.VMEM((1, H), jnp.float32)],
    )(a5, w, b.reshape(1, H), rs_d)[:, 0, :]


def _tail(m_svc, m_node, m_pod, lsw, lsb, lnw, lnb, lpw, lpb,
          wx0t, wh0t, b0, wx1t, wh1t, b1):
    def body(ms, mn, mp, lsw_r, lsb_r, lnw_r, lnb_r, lpw_r, lpb_r,
             wx0_r, wh0_r, b0_r, wx1_r, wh1_r, b1_r, o_ref, vbuf, seq0):
        vbuf[...] = (ms[...] @ lsw_r[...] + lsb_r[...]
                     + mn[...] @ lnw_r[...] + lnb_r[...]
                     + mp[...] @ lpw_r[...] + lpb_r[...]) * (1.0 / 3.0)

        def layer(src_ref, wx, wh, bb, dst_ref):
            def step(t, hc):
                h, c = hc
                z = src_ref[pl.ds(t, 1), :] @ wx + h @ wh + bb
                ii = jax.nn.sigmoid(z[:, 0:H])
                ff = jax.nn.sigmoid(z[:, H:2 * H])
                gg = jnp.tanh(z[:, 2 * H:3 * H])
                oo = jax.nn.sigmoid(z[:, 3 * H:4 * H])
                c = ff * c + ii * gg
                h = oo * jnp.tanh(c)
                dst_ref[pl.ds(t, 1), :] = h
                return (h, c)
            z0 = jnp.zeros((1, H), jnp.float32)
            lax.fori_loop(0, T, step, (z0, z0))

        layer(vbuf, wx0_r[...], wh0_r[...], b0_r[...], seq0)
        layer(seq0, wx1_r[...], wh1_r[...], b1_r[...], o_ref)

    return pl.pallas_call(
        body,
        out_shape=jax.ShapeDtypeStruct((T, H), jnp.float32),
        scratch_shapes=[pltpu.VMEM((T, H), jnp.float32),
                        pltpu.VMEM((T, H), jnp.float32)],
    )(m_svc, m_node, m_pod, lsw, lsb.reshape(1, H), lnw, lnb.reshape(1, H),
      lpw, lpb.reshape(1, H), wx0t, wh0t, b0.reshape(1, 4 * H),
      wx1t, wh1t, b1.reshape(1, 4 * H))


def kernel(svc_feat, pod_feat, node_feat, W_svc, b_svc, W_in, b_in, W_np, b_np,
           L_svc_w, L_svc_b, L_node_w, L_node_b, L_pod_w, L_pod_b,
           lstm_Wx0, lstm_Wh0, lstm_b0, lstm_Wx1, lstm_Wh1, lstm_b1,
           edge_svc_src, edge_svc_dst, edge_in_src, edge_in_dst,
           edge_np_src, edge_np_dst):
    def pad_e(src, dst, e_pad, n_dst, n_dst_pad):
        npad = e_pad - src.shape[0]
        ar = jnp.arange(npad, dtype=jnp.int32)
        return (jnp.concatenate([src, ar % 128]),
                jnp.concatenate([dst, n_dst + ar % (n_dst_pad - n_dst)]))

    svc_s, svc_d = pad_e(edge_svc_src, edge_svc_dst, EP_SVC, N_SVC, P_SVC)
    in_s, in_d = pad_e(edge_in_src, edge_in_dst, EP_IN, N_NODE, P_NODE)
    np_s, np_d = pad_e(edge_np_src, edge_np_dst, EP_NP, N_POD, P_POD)

    h_svc_s, h_svc_d, h_in_s, h_in_d, h_np_s, h_np_d = _degree_kernel(
        svc_s, svc_d, in_s, in_d, np_s, np_d)

    rs_svc_s, rs_svc_d = _deg_finalize(h_svc_s, h_svc_d, EP_SVC - E_SVC)
    rs_pod_s, rs_pod_d = _deg_finalize(h_in_s, h_np_d, EP_IN - E_IN)
    rs_node_s, rs_node_d = _deg_finalize(h_np_s, h_in_d, EP_NP - E_NP)

    xs_svc = _prescale(svc_feat, rs_svc_s, P_SVC, 128)
    xs_pod = _prescale(pod_feat, rs_pod_s, P_POD, 128)
    xs_node = _prescale(node_feat, rs_node_s, P_NODE, 32)

    a_svc = _aggregate(xs_svc, svc_s.reshape(-1, SB), svc_d.reshape(-1, SB),
                       P_SVC, P_SVC, EP_SVC, 128)
    a_node = _aggregate(xs_pod, in_s.reshape(-1, SB), in_d.reshape(-1, SB),
                        P_POD, P_NODE, EP_IN, 128)
    a_pod = _aggregate(xs_node, np_s.reshape(-1, SB), np_d.reshape(-1, SB),
                       P_NODE, P_POD, EP_NP, 32)

    m_svc = _reduce(a_svc, W_svc, b_svc, rs_svc_d, N_SVC, P_SVC, 128)
    m_node = _reduce(a_node, W_in, b_in, rs_node_d, N_NODE, P_NODE, 128)
    m_pod = _reduce(a_pod, W_np, b_np, rs_pod_d, N_POD, P_POD, 32)

    return _tail(m_svc, m_node, m_pod, L_svc_w, L_svc_b, L_node_w, L_node_b,
                 L_pod_w, L_pod_b, lstm_Wx0.T, lstm_Wh0.T, lstm_b0,
                 lstm_Wx1.T, lstm_Wh1.T, lstm_b1)


# speedup vs baseline: 4.9658x; 1.0941x over previous
import functools

import jax
import jax.numpy as jnp
from jax import lax
from jax.experimental import pallas as pl
from jax.experimental.pallas import tpu as pltpu
from jax.experimental.pallas import tpu_sc as plsc

N_SVC, N_POD, N_NODE = 10000, 50000, 5000
T, D, H = 8, 128, 128
E_SVC, E_IN, E_NP = 320000, 400000, 400000

P_SVC, P_POD, P_NODE = 10240, 50176, 5120
EP_SVC, EP_IN, EP_NP = 327680, 425984, 425984
NP_PASS = 32
CCOL = 32
NLANE = 16
NTILE = 16
NCORE = 2
SB = 128


def _sc_mesh():
    return plsc.VectorSubcoreMesh(core_axis_name="c", subcore_axis_name="s",
                                  num_cores=NCORE, num_subcores=NTILE)


def _degree_kernel(svc_s, svc_d, in_s, in_d, np_s, np_d):
    specs = [
        (svc_s, EP_SVC, P_SVC), (svc_d, EP_SVC, P_SVC),
        (in_s, EP_IN, P_POD), (in_d, EP_IN, P_NODE),
        (np_s, EP_NP, P_NODE), (np_d, EP_NP, P_POD),
    ]
    out_type = [jax.ShapeDtypeStruct((NCORE * NTILE, p), jnp.float32)
                for (_, _, p) in specs]
    scratch = [
        pltpu.VMEM((max(p for (_, _, p) in specs),), jnp.float32),
        pltpu.VMEM((max(e for (_, e, _) in specs) // (NCORE * NTILE) // 128, 128),
                   jnp.int32),
    ]

    def body(s1, s2, s3, s4, s5, s6, o1, o2, o3, o4, o5, o6, hist, idxc):
        wid = lax.axis_index("s") * NCORE + lax.axis_index("c")
        ones = jnp.ones((NLANE,), jnp.float32)
        for (arr, e_pad, nbins), out in zip(
                [(s1, EP_SVC, P_SVC), (s2, EP_SVC, P_SVC),
                 (s3, EP_IN, P_POD), (s4, EP_IN, P_NODE),
                 (s5, EP_NP, P_NODE), (s6, EP_NP, P_POD)],
                [o1, o2, o3, o4, o5, o6]):
            chunk = e_pad // (NCORE * NTILE)
            rows = chunk // 128

            def zero_b(i, _):
                hist[pl.ds(i * NLANE, NLANE)] = jnp.zeros((NLANE,), jnp.float32)
                return 0
            lax.fori_loop(0, nbins // NLANE, zero_b, 0)
            pltpu.sync_copy(arr.at[pl.ds(wid * rows, rows)], idxc.at[pl.ds(0, rows)])

            def scat(i, _):
                r = i // 8
                k = lax.rem(i, 8)
                idx = idxc[r, pl.ds(k * NLANE, NLANE)]
                plsc.addupdate_scatter(hist, [idx], ones)
                return 0
            lax.fori_loop(0, rows * 8, scat, 0)
            pltpu.sync_copy(hist.at[pl.ds(0, nbins)], out.at[wid])

    fn = pl.kernel(body, out_type=out_type, mesh=_sc_mesh(),
                   scratch_types=scratch,
                   compiler_params=pltpu.CompilerParams(
                       needs_layout_passes=False))
    return fn(svc_s.reshape(-1, 128), svc_d.reshape(-1, 128),
              in_s.reshape(-1, 128), in_d.reshape(-1, 128),
              np_s.reshape(-1, 128), np_d.reshape(-1, 128))


def _deg_finalize(hist_s, hist_d, pad_cnt):
    npd = hist_s.shape[1]
    bq = 512

    def body(hs_ref, hd_ref, rs_ref, rd_ref):
        i = pl.program_id(0)
        first = (lax.broadcasted_iota(jnp.int32, (bq,), 0) < 128) & (i == 0)
        ds = jnp.sum(hs_ref[...], axis=0)
        ds = jnp.maximum(ds - jnp.where(first, jnp.float32(pad_cnt // 128),
                                        0.0), 1.0)
        dd = jnp.maximum(jnp.sum(hd_ref[...], axis=0), 1.0)
        rs_ref[...] = jnp.broadcast_to(lax.rsqrt(ds)[:, None], (bq, 8))
        rd_ref[...] = jnp.broadcast_to(lax.rsqrt(dd)[:, None], (bq, 8))

    return pl.pallas_call(
        body,
        grid=(npd // bq,),
        in_specs=[pl.BlockSpec((NCORE * NTILE, bq), lambda i: (0, i))] * 2,
        out_specs=[pl.BlockSpec((bq, 8), lambda i: (i, 0))] * 2,
        out_shape=[jax.ShapeDtypeStruct((npd, 8), jnp.float32)] * 2,
    )(hist_s, hist_d)


def _prescale(feat, rs, npd, ccol):
    n = feat.shape[0]
    bn = 512
    nb = npd // bn
    nch = D // ccol

    def body(f_ref, rs_ref, o_ref):
        xs = (f_ref[...] * rs_ref[:, 0:1]).astype(jnp.bfloat16)
        for t in range(T):
            for c in range(nch):
                o_ref[t, c] = xs[:, t * D + c * ccol:t * D + (c + 1) * ccol]

    out = pl.pallas_call(
        body,
        grid=(nb,),
        in_specs=[
            pl.BlockSpec((bn, T * D), lambda i: (i, 0)),
            pl.BlockSpec((bn, 8), lambda i: (i, 0)),
        ],
        out_specs=pl.BlockSpec((T, nch, bn, ccol), lambda i: (0, 0, i, 0)),
        out_shape=jax.ShapeDtypeStruct((T, nch, npd, ccol), jnp.bfloat16),
    )(feat.reshape(n, T * D), rs)
    return out.reshape(T * nch * npd, ccol)


def _aggregate(xs_flat, src2d, dst2d, n_src_pad, n_dst_pad, e_pad, ccol):
    chunk = e_pad // (NCORE * NTILE)
    b2 = chunk // SB
    rpt = n_dst_pad // NTILE
    nz = rpt // 64
    n_pass = (T * D) // ccol

    def body(xs_hbm, src_hbm, dst_hbm, out_hbm, srcc, dstc, rows0, rows1,
             rows2, rows3, zbuf, acc, sem0, sem1, sem2, sem3):
        cid = lax.axis_index("c")
        sid = lax.axis_index("s")
        row0 = (cid * NTILE + sid) * b2

        def zb(i, _):
            for q in range(ccol // 32):
                zbuf[i, pl.ds(q * 32, 32)] = jnp.zeros((32,), jnp.bfloat16)
            return 0
        lax.fori_loop(0, 64, zb, 0)
        pltpu.sync_copy(src_hbm.at[pl.ds(row0, b2)], srcc)
        pltpu.sync_copy(dst_hbm.at[pl.ds(row0, b2)], dstc)

        def one_pass(p, _):
            @pl.when(p > 0)
            def _():
                def badd(i, _):
                    r = i // 8
                    k = lax.rem(i, 8)
                    srcc[r, pl.ds(k * NLANE, NLANE)] = (
                        srcc[r, pl.ds(k * NLANE, NLANE)] + n_src_pad)
                    return 0
                lax.fori_loop(0, b2 * 8, badd, 0)

            def zs(i, _):
                pltpu.sync_copy(zbuf, acc.at[pl.ds(sid * rpt + i * 64, 64)])
                return 0
            lax.fori_loop(0, nz, zs, 0)
            plsc.subcore_barrier()

            bufs = [(rows0, sem0), (rows1, sem1), (rows2, sem2),
                    (rows3, sem3)]
            for q in range(3):
                pltpu.async_copy(xs_hbm.at[srcc.at[q]], bufs[q][0],
                                 bufs[q][1])

            def burst4(j4, _):
                j = 4 * j4
                for q in range(4):
                    jj = j + q
                    rb, sm = bufs[q]
                    nb_, ns_ = bufs[(q + 3) % 4]

                    @pl.when(jj + 3 < b2)
                    def _(jj=jj, nb_=nb_, ns_=ns_):
                        pltpu.async_copy(xs_hbm.at[srcc.at[jj + 3]], nb_, ns_)

                    pltpu.make_async_copy(xs_hbm.at[srcc.at[jj]], rb,
                                          sm).wait()
                    pltpu.sync_copy(rb, acc.at[dstc.at[jj]], add=True)
                return 0
            lax.fori_loop(0, b2 // 4, burst4, 0)
            plsc.subcore_barrier()
            pltpu.sync_copy(acc.at[pl.ds(sid * rpt, rpt)],
                            out_hbm.at[cid, p, pl.ds(sid * rpt, rpt)])
            return 0
        lax.fori_loop(0, n_pass, one_pass, 0)

    fn = pl.kernel(
        body,
        out_type=jax.ShapeDtypeStruct((NCORE, n_pass, n_dst_pad, ccol),
                                      jnp.bfloat16),
        mesh=_sc_mesh(),
        scratch_types=[
            pltpu.VMEM((b2, SB), jnp.int32),
            pltpu.VMEM((b2, SB), jnp.int32),
            pltpu.VMEM((SB, ccol), jnp.bfloat16),
            pltpu.VMEM((SB, ccol), jnp.bfloat16),
            pltpu.VMEM((SB, ccol), jnp.bfloat16),
            pltpu.VMEM((SB, ccol), jnp.bfloat16),
            pltpu.VMEM((64, ccol), jnp.bfloat16),
            pltpu.VMEM_SHARED((n_dst_pad, ccol), jnp.bfloat16),
            pltpu.SemaphoreType.DMA,
            pltpu.SemaphoreType.DMA,
            pltpu.SemaphoreType.DMA,
            pltpu.SemaphoreType.DMA,
        ],
        compiler_params=pltpu.CompilerParams(needs_layout_passes=False,
                                             use_tc_tiling_on_sc=False),
    )
    return fn(xs_flat, src2d, dst2d)


def _reduce(a_part, w, b, rs_d, n_dst, n_dst_pad, ccol):
    bnd = 512
    nb = n_dst_pad // bnd
    nch = D // ccol
    a5 = a_part.reshape(NCORE, T, nch, n_dst_pad, ccol)

    def body(a_ref, w_ref, b_ref, rs_ref, o_ref, acc, macc):
        t, i, c = pl.program_id(0), pl.program_id(1), pl.program_id(2)
        part = (a_ref[0, 0, 0].astype(jnp.float32)
                + a_ref[1, 0, 0].astype(jnp.float32)) @ w_ref[...]

        @pl.when(c == 0)
        def _():
            acc[...] = part

        @pl.when(c > 0)
        def _():
            acc[...] = acc[...] + part

        @pl.when(jnp.logical_and(i == 0, c == 0))
        def _():
            macc[...] = jnp.zeros_like(macc)

        @pl.when(c == nch - 1)
        def _():
            h = acc[...] * rs_ref[:, 0:1] + b_ref[...]
            lr = jnp.where(h > 0, h, 0.01 * h)
            row = (lax.broadcasted_iota(jnp.int32, (bnd, 1), 0)
                   + i * bnd)
            lr = jnp.where(row < n_dst, lr, 0.0)
            macc[...] = macc[...] + jnp.sum(lr, axis=0, keepdims=True)

        @pl.when(jnp.logical_and(i == nb - 1, c == nch - 1))
        def _():
            o_ref[...] = jnp.broadcast_to(macc[...][:, None, :] * (1.0 / n_dst),
                                          (1, 8, H))

    return pl.pallas_call(
        body,
        grid=(T, nb, nch),
        in_specs=[
            pl.BlockSpec((NCORE, 1, 1, bnd, ccol), lambda t, i, c: (0, t, c, i, 0)),
            pl.BlockSpec((ccol, H), lambda t, i, c: (c, 0)),
            pl.BlockSpec((1, H), lambda t, i, c: (0, 0)),
            pl.BlockSpec((bnd, 8), lambda t, i, c: (i, 0)),
        ],
        out_specs=pl.BlockSpec((1, 8, H), lambda t, i, c: (t, 0, 0)),
        out_shape=jax.ShapeDtypeStruct((T, 8, H), jnp.float32),
        scratch_shapes=[pltpu.VMEM((bnd, H), jnp.float32),
                        pltpu.VMEM((1, H), jnp.float32)],
    )(a5, w, b.reshape(1, H), rs_d)[:, 0, :]


def _tail(m_svc, m_node, m_pod, lsw, lsb, lnw, lnb, lpw, lpb,
          wx0t, wh0t, b0, wx1t, wh1t, b1):
    def body(ms, mn, mp, lsw_r, lsb_r, lnw_r, lnb_r, lpw_r, lpb_r,
             wx0_r, wh0_r, b0_r, wx1_r, wh1_r, b1_r, o_ref, vbuf, seq0):
        vbuf[...] = (ms[...] @ lsw_r[...] + lsb_r[...]
                     + mn[...] @ lnw_r[...] + lnb_r[...]
                     + mp[...] @ lpw_r[...] + lpb_r[...]) * (1.0 / 3.0)

        def layer(src_ref, wx, wh, bb, dst_ref):
            def step(t, hc):
                h, c = hc
                z = src_ref[pl.ds(t, 1), :] @ wx + h @ wh + bb
                ii = jax.nn.sigmoid(z[:, 0:H])
                ff = jax.nn.sigmoid(z[:, H:2 * H])
                gg = jnp.tanh(z[:, 2 * H:3 * H])
                oo = jax.nn.sigmoid(z[:, 3 * H:4 * H])
                c = ff * c + ii * gg
                h = oo * jnp.tanh(c)
                dst_ref[pl.ds(t, 1), :] = h
                return (h, c)
            z0 = jnp.zeros((1, H), jnp.float32)
            lax.fori_loop(0, T, step, (z0, z0))

        layer(vbuf, wx0_r[...], wh0_r[...], b0_r[...], seq0)
        layer(seq0, wx1_r[...], wh1_r[...], b1_r[...], o_ref)

    return pl.pallas_call(
        body,
        out_shape=jax.ShapeDtypeStruct((T, H), jnp.float32),
        scratch_shapes=[pltpu.VMEM((T, H), jnp.float32),
                        pltpu.VMEM((T, H), jnp.float32)],
    )(m_svc, m_node, m_pod, lsw, lsb.reshape(1, H), lnw, lnb.reshape(1, H),
      lpw, lpb.reshape(1, H), wx0t, wh0t, b0.reshape(1, 4 * H),
      wx1t, wh1t, b1.reshape(1, 4 * H))


def kernel(svc_feat, pod_feat, node_feat, W_svc, b_svc, W_in, b_in, W_np, b_np,
           L_svc_w, L_svc_b, L_node_w, L_node_b, L_pod_w, L_pod_b,
           lstm_Wx0, lstm_Wh0, lstm_b0, lstm_Wx1, lstm_Wh1, lstm_b1,
           edge_svc_src, edge_svc_dst, edge_in_src, edge_in_dst,
           edge_np_src, edge_np_dst):
    def pad_e(src, dst, e_pad, n_dst, n_dst_pad):
        npad = e_pad - src.shape[0]
        ar = jnp.arange(npad, dtype=jnp.int32)
        return (jnp.concatenate([src, ar % 128]),
                jnp.concatenate([dst, n_dst + ar % (n_dst_pad - n_dst)]))

    svc_s, svc_d = pad_e(edge_svc_src, edge_svc_dst, EP_SVC, N_SVC, P_SVC)
    in_s, in_d = pad_e(edge_in_src, edge_in_dst, EP_IN, N_NODE, P_NODE)
    np_s, np_d = pad_e(edge_np_src, edge_np_dst, EP_NP, N_POD, P_POD)

    h_svc_s, h_svc_d, h_in_s, h_in_d, h_np_s, h_np_d = _degree_kernel(
        svc_s, svc_d, in_s, in_d, np_s, np_d)

    rs_svc_s, rs_svc_d = _deg_finalize(h_svc_s, h_svc_d, EP_SVC - E_SVC)
    rs_pod_s, rs_pod_d = _deg_finalize(h_in_s, h_np_d, EP_IN - E_IN)
    rs_node_s, rs_node_d = _deg_finalize(h_np_s, h_in_d, EP_NP - E_NP)

    xs_svc = _prescale(svc_feat, rs_svc_s, P_SVC, 128)
    xs_pod = _prescale(pod_feat, rs_pod_s, P_POD, 128)
    xs_node = _prescale(node_feat, rs_node_s, P_NODE, 32)

    a_svc = _aggregate(xs_svc, svc_s.reshape(-1, SB), svc_d.reshape(-1, SB),
                       P_SVC, P_SVC, EP_SVC, 128)
    a_node = _aggregate(xs_pod, in_s.reshape(-1, SB), in_d.reshape(-1, SB),
                        P_POD, P_NODE, EP_IN, 128)
    a_pod = _aggregate(xs_node, np_s.reshape(-1, SB), np_d.reshape(-1, SB),
                       P_NODE, P_POD, EP_NP, 32)

    m_svc = _reduce(a_svc, W_svc, b_svc, rs_svc_d, N_SVC, P_SVC, 128)
    m_node = _reduce(a_node, W_in, b_in, rs_node_d, N_NODE, P_NODE, 128)
    m_pod = _reduce(a_pod, W_np, b_np, rs_pod_d, N_POD, P_POD, 32)

    return _tail(m_svc, m_node, m_pod, L_svc_w, L_svc_b, L_node_w, L_node_b,
                 L_pod_w, L_pod_b, lstm_Wx0.T, lstm_Wh0.T, lstm_b0,
                 lstm_Wx1.T, lstm_Wh1.T, lstm_b1)
